# Initial kernel scaffold; baseline (speedup 1.0000x reference)
import jax
import jax.numpy as jnp
from jax.experimental import pallas as pl


def kernel(x, edge_index, edge_weight, batch, params):
    raise NotImplementedError("write your pallas kernel here")


# speedup vs baseline: 2.5038x; 2.5038x over previous
import math

import jax
import jax.numpy as jnp
from jax.experimental import pallas as pl
from jax.experimental.pallas import tpu as pltpu

_B = 64
_HEADS = 4


def _cdiv(a, b):
    return (a + b - 1) // b


def _rup(a, b):
    return _cdiv(a, b) * b


def _mm(x, y, *, bias=None, addend=None, relu=False):
    M, K = x.shape
    _, Nn = y.shape
    bm = min(256, M)
    bn = min(512, Nn)
    bk = min(512, K)
    gm, gn, gk = M // bm, Nn // bn, K // bk

    def kern(*refs):
        i = 0
        x_ref, y_ref = refs[0], refs[1]
        nxt = 2
        b_ref = ad_ref = None
        if bias is not None:
            b_ref = refs[nxt]
            nxt += 1
        if addend is not None:
            ad_ref = refs[nxt]
            nxt += 1
        o_ref, acc_ref = refs[nxt], refs[nxt + 1]
        k = pl.program_id(2)

        @pl.when(k == 0)
        def _():
            acc_ref[...] = jnp.zeros_like(acc_ref)

        acc_ref[...] += jnp.dot(x_ref[...], y_ref[...],
                                preferred_element_type=jnp.float32)

        @pl.when(k == gk - 1)
        def _():
            r = acc_ref[...]
            if b_ref is not None:
                r = r + b_ref[0:1, :]
            if ad_ref is not None:
                r = r + ad_ref[...].astype(jnp.float32)
            if relu:
                r = jnp.maximum(r, 0.0)
            o_ref[...] = r

    in_specs = [
        pl.BlockSpec((bm, bk), lambda i, j, k: (i, k)),
        pl.BlockSpec((bk, bn), lambda i, j, k: (k, j)),
    ]
    ops = [x, y]
    if bias is not None:
        in_specs.append(pl.BlockSpec((8, bn), lambda i, j, k: (0, j)))
        ops.append(bias)
    if addend is not None:
        in_specs.append(pl.BlockSpec((bm, bn), lambda i, j, k: (i, j)))
        ops.append(addend)
    return pl.pallas_call(
        kern,
        grid=(gm, gn, gk),
        in_specs=in_specs,
        out_specs=pl.BlockSpec((bm, bn), lambda i, j, k: (i, j)),
        out_shape=jax.ShapeDtypeStruct((M, Nn), jnp.float32),
        scratch_shapes=[pltpu.VMEM((bm, bn), jnp.float32)],
        compiler_params=pltpu.CompilerParams(
            dimension_semantics=("parallel", "parallel", "arbitrary")),
    )(*ops)


def _colstats(z, n_valid):
    M, Dh = z.shape
    bm = min(512, M)
    gm = M // bm

    def kern(z_ref, o_ref):
        i = pl.program_id(0)

        @pl.when(i == 0)
        def _():
            o_ref[...] = jnp.zeros_like(o_ref)

        rows = jax.lax.broadcasted_iota(jnp.int32, (bm, 1), 0) + i * bm
        zz = jnp.where(rows < n_valid, z_ref[...], 0.0)
        o_ref[0:1, :] += jnp.sum(zz, axis=0, keepdims=True)
        o_ref[1:2, :] += jnp.sum(zz * zz, axis=0, keepdims=True)

    return pl.pallas_call(
        kern,
        grid=(gm,),
        in_specs=[pl.BlockSpec((bm, Dh), lambda i: (i, 0))],
        out_specs=pl.BlockSpec((8, Dh), lambda i: (0, 0)),
        out_shape=jax.ShapeDtypeStruct((8, Dh), jnp.float32),
        compiler_params=pltpu.CompilerParams(
            dimension_semantics=("arbitrary",)),
    )(z)


def _bn_relu(z, stats, g8, b8, n_valid):
    M, Dh = z.shape
    bm = min(512, M)

    def kern(z_ref, s_ref, g_ref, b_ref, o_ref):
        inv_n = 1.0 / n_valid
        mu = s_ref[0:1, :] * inv_n
        var = s_ref[1:2, :] * inv_n - mu * mu
        scale = jax.lax.rsqrt(var + 1e-5) * g_ref[0:1, :]
        r = (z_ref[...] - mu) * scale + b_ref[0:1, :]
        o_ref[...] = jnp.maximum(r, 0.0).astype(jnp.bfloat16)

    return pl.pallas_call(
        kern,
        grid=(M // bm,),
        in_specs=[
            pl.BlockSpec((bm, Dh), lambda i: (i, 0)),
            pl.BlockSpec((8, Dh), lambda i: (0, 0)),
            pl.BlockSpec((8, Dh), lambda i: (0, 0)),
            pl.BlockSpec((8, Dh), lambda i: (0, 0)),
        ],
        out_specs=pl.BlockSpec((bm, Dh), lambda i: (i, 0)),
        out_shape=jax.ShapeDtypeStruct((M, Dh), jnp.bfloat16),
        compiler_params=pltpu.CompilerParams(
            dimension_semantics=("arbitrary",)),
    )(z, stats, g8, b8)


def _scale_dinv(v, indeg, n_valid, bias8=None, out_dtype=jnp.float32):
    M, C = v.shape
    bm = min(512, M)

    def kern(*refs):
        if bias8 is not None:
            v_ref, ind_ref, b_ref, o_ref = refs
        else:
            v_ref, ind_ref, o_ref = refs
            b_ref = None
        i = pl.program_id(0)
        rows = jax.lax.broadcasted_iota(jnp.int32, (bm, 1), 0) + i * bm
        ind = ind_ref[...][:, 0:1]
        dinv = jnp.where(rows < n_valid, jax.lax.rsqrt(ind + 1.0), 0.0)
        r = v_ref[...].astype(jnp.float32) * dinv
        if b_ref is not None:
            r = r + b_ref[0:1, :]
        o_ref[...] = r.astype(out_dtype)

    in_specs = [
        pl.BlockSpec((bm, C), lambda i: (i, 0)),
        pl.BlockSpec((bm, 128), lambda i: (i, 0)),
    ]
    ops = [v, indeg]
    if bias8 is not None:
        in_specs.append(pl.BlockSpec((8, C), lambda i: (0, 0)))
        ops.append(bias8)
    return pl.pallas_call(
        kern,
        grid=(M // bm,),
        in_specs=in_specs,
        out_specs=pl.BlockSpec((bm, C), lambda i: (i, 0)),
        out_shape=jax.ShapeDtypeStruct((M, C), out_dtype),
        compiler_params=pltpu.CompilerParams(
            dimension_semantics=("arbitrary",)),
    )(*ops)


def _pma1(kv, s1p, wq, bq8, wo, bo8, starts, n_seeds):
    Np, C = kv.shape
    D = C // 2
    hd = D // _HEADS
    Sp, _ = s1p.shape
    T = 1024 if Np % 1024 == 0 else 512
    NT = Np // T
    scale = 1.0 / math.sqrt(D)

    def kern(st_ref, kv_ref, s1_ref, wq_ref, bq_ref, wo_ref, bo_ref,
             o_ref, q_ref, m_ref, l_ref, acc_ref):
        b = pl.program_id(0)
        t = pl.program_id(1)
        start = st_ref[b]
        end = st_ref[b + 1]

        @pl.when(t == 0)
        def _():
            q_ref[...] = jnp.dot(s1_ref[...], wq_ref[...],
                                 preferred_element_type=jnp.float32) + bq_ref[0:1, :]
            m_ref[...] = jnp.full((Sp, D), -1e30, jnp.float32)
            l_ref[...] = jnp.zeros((Sp, D), jnp.float32)
            acc_ref[...] = jnp.zeros((Sp, D), jnp.float32)

        kt = kv_ref[pl.ds(t * T, T), :]
        ids = jax.lax.broadcasted_iota(jnp.int32, (1, T), 1) + t * T
        valid = (ids >= start) & (ids < end)
        qp = q_ref[...]
        for h in range(_HEADS):
            lo, hi = h * hd, (h + 1) * hd
            qh = qp[:, lo:hi]
            kh = kt[:, lo:hi]
            vh = kt[:, D + lo:D + hi]
            s = jax.lax.dot_general(qh, kh, (((1,), (1,)), ((), ())),
                                    preferred_element_type=jnp.float32) * scale
            s = jnp.where(valid, s, -1e30)
            mold = m_ref[:, lo:hi][:, 0:1]
            mnew = jnp.maximum(mold, jnp.max(s, axis=1, keepdims=True))
            p = jnp.where(valid, jnp.exp(s - mnew), 0.0)
            resc = jnp.exp(mold - mnew)
            lnew = l_ref[:, lo:hi][:, 0:1] * resc + jnp.sum(p, axis=1, keepdims=True)
            accn = acc_ref[:, lo:hi] * resc + jnp.dot(
                p, vh, preferred_element_type=jnp.float32)
            m_ref[:, lo:hi] = jnp.broadcast_to(mnew, (Sp, hd))
            l_ref[:, lo:hi] = jnp.broadcast_to(lnew, (Sp, hd))
            acc_ref[:, lo:hi] = accn

        @pl.when(t == NT - 1)
        def _():
            l = l_ref[...]
            attn = acc_ref[...] * jnp.where(l > 0, 1.0 / l, 0.0)
            o = q_ref[...] + attn
            o2 = jnp.dot(o, wo_ref[...], preferred_element_type=jnp.float32)
            o_ref[0] = o + jnp.maximum(o2 + bo_ref[0:1, :], 0.0)

    return pl.pallas_call(
        kern,
        grid=(_B, NT),
        in_specs=[
            pl.BlockSpec(memory_space=pltpu.SMEM),
            pl.BlockSpec((Np, C), lambda b, t: (0, 0)),
            pl.BlockSpec((Sp, D), lambda b, t: (0, 0)),
            pl.BlockSpec((D, D), lambda b, t: (0, 0)),
            pl.BlockSpec((8, D), lambda b, t: (0, 0)),
            pl.BlockSpec((D, D), lambda b, t: (0, 0)),
            pl.BlockSpec((8, D), lambda b, t: (0, 0)),
        ],
        out_specs=pl.BlockSpec((1, Sp, D), lambda b, t: (b, 0, 0)),
        out_shape=jax.ShapeDtypeStruct((_B, Sp, D), jnp.float32),
        scratch_shapes=[pltpu.VMEM((Sp, D), jnp.float32)] * 4,
        compiler_params=pltpu.CompilerParams(
            dimension_semantics=("arbitrary", "arbitrary")),
    )(starts, kv, s1p, wq, bq8, wo, bo8)


def _attn_block(x, kk, vv, qh_all, wo, bo8, n_seeds, D):
    hd = D // _HEADS
    scale = 1.0 / math.sqrt(D)
    Sp = kk.shape[0]
    cols = jax.lax.broadcasted_iota(jnp.int32, (1, Sp), 1)
    keymask = cols < n_seeds
    outs = []
    for h in range(_HEADS):
        lo, hi = h * hd, (h + 1) * hd
        qh = qh_all[:, lo:hi]
        kh = kk[:, lo:hi]
        vh = vv[:, lo:hi]
        s = jax.lax.dot_general(qh, kh, (((1,), (1,)), ((), ())),
                                preferred_element_type=jnp.float32) * scale
        s = jnp.where(keymask, s, -1e30)
        m = jnp.max(s, axis=1, keepdims=True)
        p = jnp.exp(s - m)
        l = jnp.sum(p, axis=1, keepdims=True)
        a = p * (1.0 / l)
        outs.append(qh + jnp.dot(a, vh, preferred_element_type=jnp.float32))
    o = jnp.concatenate(outs, axis=1)
    o2 = jnp.dot(o, wo, preferred_element_type=jnp.float32)
    return o + jnp.maximum(o2 + bo8[0:1, :], 0.0)


def _gmt_tail(bx1, n_seeds, sab, pma2, s2p, lin2w, lin2b8, outw, outb8):
    Bt, Sp, D = bx1.shape

    (swq, sbq8, swk, sbk8, swv, sbv8, swo, sbo8) = sab
    (pwq, pbq8, pwk, pbk8, pwv, pbv8, pwo, pbo8) = pma2

    def kern(x_ref, swq_r, sbq_r, swk_r, sbk_r, swv_r, sbv_r, swo_r, sbo_r,
             s2_r, pwq_r, pbq_r, pwk_r, pbk_r, pwv_r, pbv_r, pwo_r, pbo_r,
             l2w_r, l2b_r, ow_r, ob_r, o_ref):
        x = x_ref[0]
        qp = jnp.dot(x, swq_r[...], preferred_element_type=jnp.float32) + sbq_r[0:1, :]
        kk = jnp.dot(x, swk_r[...], preferred_element_type=jnp.float32) + sbk_r[0:1, :]
        vv = jnp.dot(x, swv_r[...], preferred_element_type=jnp.float32) + sbv_r[0:1, :]
        x2 = _attn_block(x, kk, vv, qp, swo_r[...], sbo_r[...], n_seeds, D)
        q2 = jnp.dot(s2_r[...], pwq_r[...], preferred_element_type=jnp.float32) + pbq_r[0:1, :]
        k2 = jnp.dot(x2, pwk_r[...], preferred_element_type=jnp.float32) + pbk_r[0:1, :]
        v2 = jnp.dot(x2, pwv_r[...], preferred_element_type=jnp.float32) + pbv_r[0:1, :]
        x3 = _attn_block(None, k2, v2, q2, pwo_r[...], pbo_r[...], n_seeds, D)
        gx = jnp.dot(x3, l2w_r[...], preferred_element_type=jnp.float32) + l2b_r[0:1, :]
        y = jnp.dot(gx, ow_r[...], preferred_element_type=jnp.float32) + ob_r[0:1, :]
        o_ref[0] = y

    wb = lambda shape: pl.BlockSpec(shape, lambda b: (0, 0))
    return pl.pallas_call(
        kern,
        grid=(Bt,),
        in_specs=[pl.BlockSpec((1, Sp, D), lambda b: (b, 0, 0))]
        + [wb((D, D)), wb((8, D))] * 4
        + [wb((8, D))]
        + [wb((D, D)), wb((8, D))] * 4
        + [wb((D, D)), wb((8, D)), wb((D, D)), wb((8, D))],
        out_specs=pl.BlockSpec((1, 8, D), lambda b: (b, 0, 0)),
        out_shape=jax.ShapeDtypeStruct((Bt, 8, D), jnp.float32),
        compiler_params=pltpu.CompilerParams(
            dimension_semantics=("arbitrary",)),
    )(bx1, swq, sbq8, swk, sbk8, swv, sbv8, swo, sbo8,
      s2p, pwq, pbq8, pwk, pbk8, pwv, pbv8, pwo, pbo8,
      lin2w, lin2b8, outw, outb8)


def _row8(b):
    return jnp.broadcast_to(b[None, :], (8, b.shape[0])).astype(jnp.float32)


def kernel(x, edge_index, edge_weight, batch, params):
    del edge_weight
    N, D = x.shape
    Np = _rup(N, 2048)
    bf = jnp.bfloat16
    src, dst = edge_index[0], edge_index[1]

    A = jnp.zeros((Np, Np), bf).at[dst, src].add(jnp.ones((), bf))

    h_bf = jnp.pad(x, ((0, Np - N), (0, 0))).astype(bf)
    add_prev = jnp.pad(x, ((0, Np - N), (0, 0)))
    for lyr in params["gin"]:
        w1, b1 = lyr["lin1"]["W"], lyr["lin1"]["b"]
        w2, b2 = lyr["lin2"]["W"], lyr["lin2"]["b"]
        h2in = _mm(A, h_bf, addend=add_prev)
        t = _mm(h2in.astype(bf), w1.astype(bf), bias=_row8(b1), relu=True)
        z = _mm(t.astype(bf), w2.astype(bf), bias=_row8(b2))
        stats = _colstats(z, N)
        h_bf = _bn_relu(z, stats, _row8(lyr["bn_g"]), _row8(lyr["bn_b"]), N)
        add_prev = h_bf

    h128 = _mm(h_bf, params["proj"]["W"].astype(bf), bias=_row8(params["proj"]["b"]))
    g = params["gmt"]
    hx = _mm(h128.astype(bf), g["lin1"]["W"].astype(bf), bias=_row8(g["lin1"]["b"]))
    hx_bf = hx.astype(bf)

    indeg = _mm(A, jnp.ones((Np, 128), bf))
    p1 = g["pma1"]
    wkv = jnp.concatenate([p1["gcn_k"]["W"], p1["gcn_v"]["W"]], axis=1)
    bkv = jnp.concatenate([p1["gcn_k"]["b"], p1["gcn_v"]["b"]])
    hkv = _mm(hx_bf, wkv.astype(bf))
    u = _scale_dinv(hkv, indeg, N, out_dtype=bf)
    w_agg = _mm(A, u, addend=u)
    kv = _scale_dinv(w_agg, indeg, N, bias8=_row8(bkv))

    starts = jnp.searchsorted(
        batch, jnp.arange(_B + 1, dtype=batch.dtype), side="left").astype(jnp.int32)
    starts = jnp.pad(starts, (0, 72 - (_B + 1)))

    S1 = p1["S"][0]
    n_seeds = S1.shape[0]
    Sp = _rup(n_seeds, 8)
    s1p = jnp.pad(S1, ((0, Sp - n_seeds), (0, 0)))
    bx1 = _pma1(kv, s1p, p1["fc_q"]["W"], _row8(p1["fc_q"]["b"]),
                p1["fc_o"]["W"], _row8(p1["fc_o"]["b"]), starts, n_seeds)

    s = g["sab"]
    p2 = g["pma2"]
    s2p = jnp.pad(p2["S"][0], ((0, 8 - p2["S"].shape[1]), (0, 0)))
    outw = jnp.pad(params["out"]["W"], ((0, 0), (0, D - params["out"]["W"].shape[1])))
    outb = jnp.pad(params["out"]["b"], (0, D - params["out"]["b"].shape[0]))
    y = _gmt_tail(
        bx1, n_seeds,
        (s["fc_q"]["W"], _row8(s["fc_q"]["b"]),
         s["layer_k"]["W"], _row8(s["layer_k"]["b"]),
         s["layer_v"]["W"], _row8(s["layer_v"]["b"]),
         s["fc_o"]["W"], _row8(s["fc_o"]["b"])),
        (p2["fc_q"]["W"], _row8(p2["fc_q"]["b"]),
         p2["layer_k"]["W"], _row8(p2["layer_k"]["b"]),
         p2["layer_v"]["W"], _row8(p2["layer_v"]["b"]),
         p2["fc_o"]["W"], _row8(p2["fc_o"]["b"])),
        s2p, g["lin2"]["W"], _row8(g["lin2"]["b"]), outw, _row8(outb))
    return y[:, 0, :params["out"]["W"].shape[1]]


# speedup vs baseline: 3.4135x; 1.3633x over previous
import math

import jax
import jax.numpy as jnp
from jax.experimental import pallas as pl
from jax.experimental.pallas import tpu as pltpu

_B = 64
_HEADS = 4


def _cdiv(a, b):
    return (a + b - 1) // b


def _rup(a, b):
    return _cdiv(a, b) * b


def _mm(x, y, *, bias=None, addend=None, relu=False, out_dtype=jnp.float32):
    M, K = x.shape
    _, Nn = y.shape
    bm = 512 if M % 512 == 0 else min(256, M)
    bn = min(1024, Nn)
    bk = min(512, K)
    gm, gn, gk = M // bm, Nn // bn, K // bk

    def kern(*refs):
        i = 0
        x_ref, y_ref = refs[0], refs[1]
        nxt = 2
        b_ref = ad_ref = None
        if bias is not None:
            b_ref = refs[nxt]
            nxt += 1
        if addend is not None:
            ad_ref = refs[nxt]
            nxt += 1
        o_ref, acc_ref = refs[nxt], refs[nxt + 1]
        k = pl.program_id(2)

        @pl.when(k == 0)
        def _():
            acc_ref[...] = jnp.zeros_like(acc_ref)

        acc_ref[...] += jnp.dot(x_ref[...], y_ref[...],
                                preferred_element_type=jnp.float32)

        @pl.when(k == gk - 1)
        def _():
            r = acc_ref[...]
            if b_ref is not None:
                r = r + b_ref[0:1, :]
            if ad_ref is not None:
                r = r + ad_ref[...].astype(jnp.float32)
            if relu:
                r = jnp.maximum(r, 0.0)
            o_ref[...] = r.astype(out_dtype)

    in_specs = [
        pl.BlockSpec((bm, bk), lambda i, j, k: (i, k)),
        pl.BlockSpec((bk, bn), lambda i, j, k: (k, j)),
    ]
    ops = [x, y]
    if bias is not None:
        in_specs.append(pl.BlockSpec((8, bn), lambda i, j, k: (0, j)))
        ops.append(bias)
    if addend is not None:
        in_specs.append(pl.BlockSpec((bm, bn), lambda i, j, k: (i, j)))
        ops.append(addend)
    return pl.pallas_call(
        kern,
        grid=(gm, gn, gk),
        in_specs=in_specs,
        out_specs=pl.BlockSpec((bm, bn), lambda i, j, k: (i, j)),
        out_shape=jax.ShapeDtypeStruct((M, Nn), out_dtype),
        scratch_shapes=[pltpu.VMEM((bm, bn), jnp.float32)],
        compiler_params=pltpu.CompilerParams(
            dimension_semantics=("parallel", "parallel", "arbitrary")),
    )(*ops)


def _colstats(z, n_valid):
    M, Dh = z.shape
    bm = min(512, M)
    gm = M // bm

    def kern(z_ref, o_ref):
        i = pl.program_id(0)

        @pl.when(i == 0)
        def _():
            o_ref[...] = jnp.zeros_like(o_ref)

        rows = jax.lax.broadcasted_iota(jnp.int32, (bm, 1), 0) + i * bm
        zz = jnp.where(rows < n_valid, z_ref[...], 0.0)
        o_ref[0:1, :] += jnp.sum(zz, axis=0, keepdims=True)
        o_ref[1:2, :] += jnp.sum(zz * zz, axis=0, keepdims=True)

    return pl.pallas_call(
        kern,
        grid=(gm,),
        in_specs=[pl.BlockSpec((bm, Dh), lambda i: (i, 0))],
        out_specs=pl.BlockSpec((8, Dh), lambda i: (0, 0)),
        out_shape=jax.ShapeDtypeStruct((8, Dh), jnp.float32),
        compiler_params=pltpu.CompilerParams(
            dimension_semantics=("arbitrary",)),
    )(z)


def _bn_relu(z, stats, g8, b8, n_valid):
    M, Dh = z.shape
    bm = min(512, M)

    def kern(z_ref, s_ref, g_ref, b_ref, o_ref):
        inv_n = 1.0 / n_valid
        mu = s_ref[0:1, :] * inv_n
        var = s_ref[1:2, :] * inv_n - mu * mu
        scale = jax.lax.rsqrt(var + 1e-5) * g_ref[0:1, :]
        r = (z_ref[...] - mu) * scale + b_ref[0:1, :]
        o_ref[...] = jnp.maximum(r, 0.0).astype(jnp.bfloat16)

    return pl.pallas_call(
        kern,
        grid=(M // bm,),
        in_specs=[
            pl.BlockSpec((bm, Dh), lambda i: (i, 0)),
            pl.BlockSpec((8, Dh), lambda i: (0, 0)),
            pl.BlockSpec((8, Dh), lambda i: (0, 0)),
            pl.BlockSpec((8, Dh), lambda i: (0, 0)),
        ],
        out_specs=pl.BlockSpec((bm, Dh), lambda i: (i, 0)),
        out_shape=jax.ShapeDtypeStruct((M, Dh), jnp.bfloat16),
        compiler_params=pltpu.CompilerParams(
            dimension_semantics=("arbitrary",)),
    )(z, stats, g8, b8)


def _scale_dinv(v, indeg, n_valid, bias8=None, out_dtype=jnp.float32):
    M, C = v.shape
    bm = min(512, M)

    def kern(*refs):
        if bias8 is not None:
            v_ref, ind_ref, b_ref, o_ref = refs
        else:
            v_ref, ind_ref, o_ref = refs
            b_ref = None
        i = pl.program_id(0)
        rows = jax.lax.broadcasted_iota(jnp.int32, (bm, 1), 0) + i * bm
        ind = ind_ref[...][:, 0:1]
        dinv = jnp.where(rows < n_valid, jax.lax.rsqrt(ind + 1.0), 0.0)
        r = v_ref[...].astype(jnp.float32) * dinv
        if b_ref is not None:
            r = r + b_ref[0:1, :]
        o_ref[...] = r.astype(out_dtype)

    in_specs = [
        pl.BlockSpec((bm, C), lambda i: (i, 0)),
        pl.BlockSpec((bm, 128), lambda i: (i, 0)),
    ]
    ops = [v, indeg]
    if bias8 is not None:
        in_specs.append(pl.BlockSpec((8, C), lambda i: (0, 0)))
        ops.append(bias8)
    return pl.pallas_call(
        kern,
        grid=(M // bm,),
        in_specs=in_specs,
        out_specs=pl.BlockSpec((bm, C), lambda i: (i, 0)),
        out_shape=jax.ShapeDtypeStruct((M, C), out_dtype),
        compiler_params=pltpu.CompilerParams(
            dimension_semantics=("arbitrary",)),
    )(*ops)


def _pma1(kv, s1p, wq, bq8, wo, bo8, starts, n_seeds):
    Np, C = kv.shape
    D = C // 2
    hd = D // _HEADS
    Sp, _ = s1p.shape
    T = 1024 if Np % 1024 == 0 else 512
    NT = Np // T
    scale = 1.0 / math.sqrt(D)

    def kern(st_ref, kv_ref, s1_ref, wq_ref, bq_ref, wo_ref, bo_ref,
             o_ref, q_ref, m_ref, l_ref, acc_ref):
        b = pl.program_id(0)
        t = pl.program_id(1)
        start = st_ref[b]
        end = st_ref[b + 1]

        @pl.when(t == 0)
        def _():
            q_ref[...] = jnp.dot(s1_ref[...], wq_ref[...],
                                 preferred_element_type=jnp.float32) + bq_ref[0:1, :]
            m_ref[...] = jnp.full((Sp, D), -1e30, jnp.float32)
            l_ref[...] = jnp.zeros((Sp, D), jnp.float32)
            acc_ref[...] = jnp.zeros((Sp, D), jnp.float32)

        kt = kv_ref[pl.ds(t * T, T), :]
        ids = jax.lax.broadcasted_iota(jnp.int32, (1, T), 1) + t * T
        valid = (ids >= start) & (ids < end)
        qp = q_ref[...]
        for h in range(_HEADS):
            lo, hi = h * hd, (h + 1) * hd
            qh = qp[:, lo:hi].astype(jnp.bfloat16)
            kh = kt[:, lo:hi].astype(jnp.bfloat16)
            vh = kt[:, D + lo:D + hi].astype(jnp.bfloat16)
            s = jax.lax.dot_general(qh, kh, (((1,), (1,)), ((), ())),
                                    preferred_element_type=jnp.float32) * scale
            s = jnp.where(valid, s, -1e30)
            mold = m_ref[:, lo:hi][:, 0:1]
            mnew = jnp.maximum(mold, jnp.max(s, axis=1, keepdims=True))
            p = jnp.where(valid, jnp.exp(s - mnew), 0.0)
            resc = jnp.exp(mold - mnew)
            lnew = l_ref[:, lo:hi][:, 0:1] * resc + jnp.sum(p, axis=1, keepdims=True)
            accn = acc_ref[:, lo:hi] * resc + jnp.dot(
                p.astype(jnp.bfloat16), vh, preferred_element_type=jnp.float32)
            m_ref[:, lo:hi] = jnp.broadcast_to(mnew, (Sp, hd))
            l_ref[:, lo:hi] = jnp.broadcast_to(lnew, (Sp, hd))
            acc_ref[:, lo:hi] = accn

        @pl.when(t == NT - 1)
        def _():
            l = l_ref[...]
            attn = acc_ref[...] * jnp.where(l > 0, 1.0 / l, 0.0)
            o = q_ref[...] + attn
            o2 = jnp.dot(o, wo_ref[...], preferred_element_type=jnp.float32)
            o_ref[0] = o + jnp.maximum(o2 + bo_ref[0:1, :], 0.0)

    return pl.pallas_call(
        kern,
        grid=(_B, NT),
        in_specs=[
            pl.BlockSpec(memory_space=pltpu.SMEM),
            pl.BlockSpec((Np, C), lambda b, t: (0, 0)),
            pl.BlockSpec((Sp, D), lambda b, t: (0, 0)),
            pl.BlockSpec((D, D), lambda b, t: (0, 0)),
            pl.BlockSpec((8, D), lambda b, t: (0, 0)),
            pl.BlockSpec((D, D), lambda b, t: (0, 0)),
            pl.BlockSpec((8, D), lambda b, t: (0, 0)),
        ],
        out_specs=pl.BlockSpec((1, Sp, D), lambda b, t: (b, 0, 0)),
        out_shape=jax.ShapeDtypeStruct((_B, Sp, D), jnp.float32),
        scratch_shapes=[pltpu.VMEM((Sp, D), jnp.float32)] * 4,
        compiler_params=pltpu.CompilerParams(
            dimension_semantics=("arbitrary", "arbitrary")),
    )(starts, kv, s1p, wq, bq8, wo, bo8)


def _attn_block(x, kk, vv, qh_all, wo, bo8, n_seeds, D):
    hd = D // _HEADS
    scale = 1.0 / math.sqrt(D)
    Sp = kk.shape[0]
    cols = jax.lax.broadcasted_iota(jnp.int32, (1, Sp), 1)
    keymask = cols < n_seeds
    outs = []
    for h in range(_HEADS):
        lo, hi = h * hd, (h + 1) * hd
        qh = qh_all[:, lo:hi]
        kh = kk[:, lo:hi]
        vh = vv[:, lo:hi]
        s = jax.lax.dot_general(qh.astype(jnp.bfloat16), kh.astype(jnp.bfloat16),
                                (((1,), (1,)), ((), ())),
                                preferred_element_type=jnp.float32) * scale
        s = jnp.where(keymask, s, -1e30)
        m = jnp.max(s, axis=1, keepdims=True)
        p = jnp.exp(s - m)
        l = jnp.sum(p, axis=1, keepdims=True)
        a = p * (1.0 / l)
        outs.append(qh + jnp.dot(a.astype(jnp.bfloat16), vh.astype(jnp.bfloat16),
                                 preferred_element_type=jnp.float32))
    o = jnp.concatenate(outs, axis=1)
    o2 = jnp.dot(o, wo, preferred_element_type=jnp.float32)
    return o + jnp.maximum(o2 + bo8[0:1, :], 0.0)


def _gmt_tail(bx1, n_seeds, sab, pma2, s2p, lin2w, lin2b8, outw, outb8):
    Bt, Sp, D = bx1.shape

    (swq, sbq8, swk, sbk8, swv, sbv8, swo, sbo8) = sab
    (pwq, pbq8, pwk, pbk8, pwv, pbv8, pwo, pbo8) = pma2

    def kern(x_ref, swq_r, sbq_r, swk_r, sbk_r, swv_r, sbv_r, swo_r, sbo_r,
             s2_r, pwq_r, pbq_r, pwk_r, pbk_r, pwv_r, pbv_r, pwo_r, pbo_r,
             l2w_r, l2b_r, ow_r, ob_r, o_ref):
        x = x_ref[0]
        qp = jnp.dot(x, swq_r[...], preferred_element_type=jnp.float32) + sbq_r[0:1, :]
        kk = jnp.dot(x, swk_r[...], preferred_element_type=jnp.float32) + sbk_r[0:1, :]
        vv = jnp.dot(x, swv_r[...], preferred_element_type=jnp.float32) + sbv_r[0:1, :]
        x2 = _attn_block(x, kk, vv, qp, swo_r[...], sbo_r[...], n_seeds, D)
        q2 = jnp.dot(s2_r[...], pwq_r[...], preferred_element_type=jnp.float32) + pbq_r[0:1, :]
        k2 = jnp.dot(x2, pwk_r[...], preferred_element_type=jnp.float32) + pbk_r[0:1, :]
        v2 = jnp.dot(x2, pwv_r[...], preferred_element_type=jnp.float32) + pbv_r[0:1, :]
        x3 = _attn_block(None, k2, v2, q2, pwo_r[...], pbo_r[...], n_seeds, D)
        gx = jnp.dot(x3, l2w_r[...], preferred_element_type=jnp.float32) + l2b_r[0:1, :]
        y = jnp.dot(gx, ow_r[...], preferred_element_type=jnp.float32) + ob_r[0:1, :]
        o_ref[0] = y

    wb = lambda shape: pl.BlockSpec(shape, lambda b: (0, 0))
    return pl.pallas_call(
        kern,
        grid=(Bt,),
        in_specs=[pl.BlockSpec((1, Sp, D), lambda b: (b, 0, 0))]
        + [wb((D, D)), wb((8, D))] * 4
        + [wb((8, D))]
        + [wb((D, D)), wb((8, D))] * 4
        + [wb((D, D)), wb((8, D)), wb((D, D)), wb((8, D))],
        out_specs=pl.BlockSpec((1, 8, D), lambda b: (b, 0, 0)),
        out_shape=jax.ShapeDtypeStruct((Bt, 8, D), jnp.float32),
        compiler_params=pltpu.CompilerParams(
            dimension_semantics=("arbitrary",)),
    )(bx1, swq, sbq8, swk, sbk8, swv, sbv8, swo, sbo8,
      s2p, pwq, pbq8, pwk, pbk8, pwv, pbv8, pwo, pbo8,
      lin2w, lin2b8, outw, outb8)


def _row8(b):
    return jnp.broadcast_to(b[None, :], (8, b.shape[0])).astype(jnp.float32)


def kernel(x, edge_index, edge_weight, batch, params):
    del edge_weight
    N, D = x.shape
    Np = _rup(N, 2048)
    bf = jnp.bfloat16
    src, dst = edge_index[0], edge_index[1]

    A = jnp.zeros((Np, Np), bf).at[dst, src].add(jnp.ones((), bf))

    h_bf = jnp.pad(x, ((0, Np - N), (0, 0))).astype(bf)
    add_prev = jnp.pad(x, ((0, Np - N), (0, 0)))
    for lyr in params["gin"]:
        w1, b1 = lyr["lin1"]["W"], lyr["lin1"]["b"]
        w2, b2 = lyr["lin2"]["W"], lyr["lin2"]["b"]
        h2in = _mm(A, h_bf, addend=add_prev, out_dtype=bf)
        t = _mm(h2in, w1.astype(bf), bias=_row8(b1), relu=True, out_dtype=bf)
        z = _mm(t, w2.astype(bf), bias=_row8(b2))
        stats = _colstats(z, N)
        h_bf = _bn_relu(z, stats, _row8(lyr["bn_g"]), _row8(lyr["bn_b"]), N)
        add_prev = h_bf

    h128 = _mm(h_bf, params["proj"]["W"].astype(bf),
               bias=_row8(params["proj"]["b"]), out_dtype=bf)
    g = params["gmt"]
    hx_bf = _mm(h128, g["lin1"]["W"].astype(bf),
                bias=_row8(g["lin1"]["b"]), out_dtype=bf)

    indeg = _mm(A, jnp.ones((Np, 128), bf))
    p1 = g["pma1"]
    wkv = jnp.concatenate([p1["gcn_k"]["W"], p1["gcn_v"]["W"]], axis=1)
    bkv = jnp.concatenate([p1["gcn_k"]["b"], p1["gcn_v"]["b"]])
    hkv = _mm(hx_bf, wkv.astype(bf))
    u = _scale_dinv(hkv, indeg, N, out_dtype=bf)
    w_agg = _mm(A, u, addend=u)
    kv = _scale_dinv(w_agg, indeg, N, bias8=_row8(bkv))

    starts = jnp.searchsorted(
        batch, jnp.arange(_B + 1, dtype=batch.dtype), side="left").astype(jnp.int32)
    starts = jnp.pad(starts, (0, 72 - (_B + 1)))

    S1 = p1["S"][0]
    n_seeds = S1.shape[0]
    Sp = _rup(n_seeds, 8)
    s1p = jnp.pad(S1, ((0, Sp - n_seeds), (0, 0)))
    bx1 = _pma1(kv, s1p, p1["fc_q"]["W"], _row8(p1["fc_q"]["b"]),
                p1["fc_o"]["W"], _row8(p1["fc_o"]["b"]), starts, n_seeds)

    s = g["sab"]
    p2 = g["pma2"]
    s2p = jnp.pad(p2["S"][0], ((0, 8 - p2["S"].shape[1]), (0, 0)))
    outw = jnp.pad(params["out"]["W"], ((0, 0), (0, D - params["out"]["W"].shape[1])))
    outb = jnp.pad(params["out"]["b"], (0, D - params["out"]["b"].shape[0]))
    y = _gmt_tail(
        bx1, n_seeds,
        (s["fc_q"]["W"], _row8(s["fc_q"]["b"]),
         s["layer_k"]["W"], _row8(s["layer_k"]["b"]),
         s["layer_v"]["W"], _row8(s["layer_v"]["b"]),
         s["fc_o"]["W"], _row8(s["fc_o"]["b"])),
        (p2["fc_q"]["W"], _row8(p2["fc_q"]["b"]),
         p2["layer_k"]["W"], _row8(p2["layer_k"]["b"]),
         p2["layer_v"]["W"], _row8(p2["layer_v"]["b"]),
         p2["fc_o"]["W"], _row8(p2["fc_o"]["b"])),
        s2p, g["lin2"]["W"], _row8(g["lin2"]["b"]), outw, _row8(outb))
    return y[:, 0, :params["out"]["W"].shape[1]]


# speedup vs baseline: 4.7133x; 1.3808x over previous
import math

import jax
import jax.numpy as jnp
from jax.experimental import pallas as pl
from jax.experimental.pallas import tpu as pltpu

_B = 64
_HEADS = 4


def _cdiv(a, b):
    return (a + b - 1) // b


def _rup(a, b):
    return _cdiv(a, b) * b


def _mm(x, y, *, bias=None, addend=None, relu=False, out_dtype=jnp.float32):
    M, K = x.shape
    _, Nn = y.shape
    bm = 512 if M % 512 == 0 else min(256, M)
    bn = min(1024, Nn)
    bk = min(512, K)
    gm, gn, gk = M // bm, Nn // bn, K // bk

    def kern(*refs):
        i = 0
        x_ref, y_ref = refs[0], refs[1]
        nxt = 2
        b_ref = ad_ref = None
        if bias is not None:
            b_ref = refs[nxt]
            nxt += 1
        if addend is not None:
            ad_ref = refs[nxt]
            nxt += 1
        o_ref, acc_ref = refs[nxt], refs[nxt + 1]
        k = pl.program_id(2)

        @pl.when(k == 0)
        def _():
            acc_ref[...] = jnp.zeros_like(acc_ref)

        acc_ref[...] += jnp.dot(x_ref[...], y_ref[...],
                                preferred_element_type=jnp.float32)

        @pl.when(k == gk - 1)
        def _():
            r = acc_ref[...]
            if b_ref is not None:
                r = r + b_ref[0:1, :]
            if ad_ref is not None:
                r = r + ad_ref[...].astype(jnp.float32)
            if relu:
                r = jnp.maximum(r, 0.0)
            o_ref[...] = r.astype(out_dtype)

    in_specs = [
        pl.BlockSpec((bm, bk), lambda i, j, k: (i, k)),
        pl.BlockSpec((bk, bn), lambda i, j, k: (k, j)),
    ]
    ops = [x, y]
    if bias is not None:
        in_specs.append(pl.BlockSpec((8, bn), lambda i, j, k: (0, j)))
        ops.append(bias)
    if addend is not None:
        in_specs.append(pl.BlockSpec((bm, bn), lambda i, j, k: (i, j)))
        ops.append(addend)
    return pl.pallas_call(
        kern,
        grid=(gm, gn, gk),
        in_specs=in_specs,
        out_specs=pl.BlockSpec((bm, bn), lambda i, j, k: (i, j)),
        out_shape=jax.ShapeDtypeStruct((M, Nn), out_dtype),
        scratch_shapes=[pltpu.VMEM((bm, bn), jnp.float32)],
        compiler_params=pltpu.CompilerParams(
            dimension_semantics=("parallel", "parallel", "arbitrary")),
    )(*ops)


def _colstats(z, n_valid):
    M, Dh = z.shape
    bm = min(512, M)
    gm = M // bm

    def kern(z_ref, o_ref):
        i = pl.program_id(0)

        @pl.when(i == 0)
        def _():
            o_ref[...] = jnp.zeros_like(o_ref)

        rows = jax.lax.broadcasted_iota(jnp.int32, (bm, 1), 0) + i * bm
        zz = jnp.where(rows < n_valid, z_ref[...], 0.0)
        o_ref[0:1, :] += jnp.sum(zz, axis=0, keepdims=True)
        o_ref[1:2, :] += jnp.sum(zz * zz, axis=0, keepdims=True)

    return pl.pallas_call(
        kern,
        grid=(gm,),
        in_specs=[pl.BlockSpec((bm, Dh), lambda i: (i, 0))],
        out_specs=pl.BlockSpec((8, Dh), lambda i: (0, 0)),
        out_shape=jax.ShapeDtypeStruct((8, Dh), jnp.float32),
        compiler_params=pltpu.CompilerParams(
            dimension_semantics=("arbitrary",)),
    )(z)


def _bn_relu(z, stats, g8, b8, n_valid):
    M, Dh = z.shape
    bm = min(512, M)

    def kern(z_ref, s_ref, g_ref, b_ref, o_ref):
        inv_n = 1.0 / n_valid
        mu = s_ref[0:1, :] * inv_n
        var = s_ref[1:2, :] * inv_n - mu * mu
        scale = jax.lax.rsqrt(var + 1e-5) * g_ref[0:1, :]
        r = (z_ref[...] - mu) * scale + b_ref[0:1, :]
        o_ref[...] = jnp.maximum(r, 0.0).astype(jnp.bfloat16)

    return pl.pallas_call(
        kern,
        grid=(M // bm,),
        in_specs=[
            pl.BlockSpec((bm, Dh), lambda i: (i, 0)),
            pl.BlockSpec((8, Dh), lambda i: (0, 0)),
            pl.BlockSpec((8, Dh), lambda i: (0, 0)),
            pl.BlockSpec((8, Dh), lambda i: (0, 0)),
        ],
        out_specs=pl.BlockSpec((bm, Dh), lambda i: (i, 0)),
        out_shape=jax.ShapeDtypeStruct((M, Dh), jnp.bfloat16),
        compiler_params=pltpu.CompilerParams(
            dimension_semantics=("arbitrary",)),
    )(z, stats, g8, b8)


def _scale_dinv(v, indeg, n_valid, bias8=None, out_dtype=jnp.float32):
    M, C = v.shape
    bm = min(512, M)

    def kern(*refs):
        if bias8 is not None:
            v_ref, ind_ref, b_ref, o_ref = refs
        else:
            v_ref, ind_ref, o_ref = refs
            b_ref = None
        i = pl.program_id(0)
        rows = jax.lax.broadcasted_iota(jnp.int32, (bm, 1), 0) + i * bm
        ind = ind_ref[...][:, 0:1].astype(jnp.float32)
        dinv = jnp.where(rows < n_valid, jax.lax.rsqrt(ind + 1.0), 0.0)
        r = v_ref[...].astype(jnp.float32) * dinv
        if b_ref is not None:
            r = r + b_ref[0:1, :]
        o_ref[...] = r.astype(out_dtype)

    in_specs = [
        pl.BlockSpec((bm, C), lambda i: (i, 0)),
        pl.BlockSpec((bm, 128), lambda i: (i, 0)),
    ]
    ops = [v, indeg]
    if bias8 is not None:
        in_specs.append(pl.BlockSpec((8, C), lambda i: (0, 0)))
        ops.append(bias8)
    return pl.pallas_call(
        kern,
        grid=(M // bm,),
        in_specs=in_specs,
        out_specs=pl.BlockSpec((bm, C), lambda i: (i, 0)),
        out_shape=jax.ShapeDtypeStruct((M, C), out_dtype),
        compiler_params=pltpu.CompilerParams(
            dimension_semantics=("arbitrary",)),
    )(*ops)


def _pma1(kv, s1p, wq, bq8, wo, bo8, starts, n_seeds):
    Np, C = kv.shape
    D = C // 2
    hd = D // _HEADS
    Sp, _ = s1p.shape
    T = 1024 if Np % 1024 == 0 else 512
    NT = Np // T
    scale = 1.0 / math.sqrt(D)

    def kern(st_ref, kv_ref, s1_ref, wq_ref, bq_ref, wo_ref, bo_ref,
             o_ref, q_ref, m_ref, l_ref, acc_ref):
        b = pl.program_id(0)
        t = pl.program_id(1)
        start = st_ref[b]
        end = st_ref[b + 1]

        @pl.when(t == 0)
        def _():
            q_ref[...] = jnp.dot(s1_ref[...], wq_ref[...],
                                 preferred_element_type=jnp.float32) + bq_ref[0:1, :]
            m_ref[...] = jnp.full((Sp, D), -1e30, jnp.float32)
            l_ref[...] = jnp.zeros((Sp, D), jnp.float32)
            acc_ref[...] = jnp.zeros((Sp, D), jnp.float32)

        kt = kv_ref[pl.ds(t * T, T), :]
        ids = jax.lax.broadcasted_iota(jnp.int32, (1, T), 1) + t * T
        valid = (ids >= start) & (ids < end)
        qp = q_ref[...]
        for h in range(_HEADS):
            lo, hi = h * hd, (h + 1) * hd
            qh = qp[:, lo:hi].astype(jnp.bfloat16)
            kh = kt[:, lo:hi].astype(jnp.bfloat16)
            vh = kt[:, D + lo:D + hi].astype(jnp.bfloat16)
            s = jax.lax.dot_general(qh, kh, (((1,), (1,)), ((), ())),
                                    preferred_element_type=jnp.float32) * scale
            s = jnp.where(valid, s, -1e30)
            mold = m_ref[:, lo:hi][:, 0:1]
            mnew = jnp.maximum(mold, jnp.max(s, axis=1, keepdims=True))
            p = jnp.where(valid, jnp.exp(s - mnew), 0.0)
            resc = jnp.exp(mold - mnew)
            lnew = l_ref[:, lo:hi][:, 0:1] * resc + jnp.sum(p, axis=1, keepdims=True)
            accn = acc_ref[:, lo:hi] * resc + jnp.dot(
                p.astype(jnp.bfloat16), vh, preferred_element_type=jnp.float32)
            m_ref[:, lo:hi] = jnp.broadcast_to(mnew, (Sp, hd))
            l_ref[:, lo:hi] = jnp.broadcast_to(lnew, (Sp, hd))
            acc_ref[:, lo:hi] = accn

        @pl.when(t == NT - 1)
        def _():
            l = l_ref[...]
            attn = acc_ref[...] * jnp.where(l > 0, 1.0 / l, 0.0)
            o = q_ref[...] + attn
            o2 = jnp.dot(o, wo_ref[...], preferred_element_type=jnp.float32)
            o_ref[0] = o + jnp.maximum(o2 + bo_ref[0:1, :], 0.0)

    return pl.pallas_call(
        kern,
        grid=(_B, NT),
        in_specs=[
            pl.BlockSpec(memory_space=pltpu.SMEM),
            pl.BlockSpec((Np, C), lambda b, t: (0, 0)),
            pl.BlockSpec((Sp, D), lambda b, t: (0, 0)),
            pl.BlockSpec((D, D), lambda b, t: (0, 0)),
            pl.BlockSpec((8, D), lambda b, t: (0, 0)),
            pl.BlockSpec((D, D), lambda b, t: (0, 0)),
            pl.BlockSpec((8, D), lambda b, t: (0, 0)),
        ],
        out_specs=pl.BlockSpec((1, Sp, D), lambda b, t: (b, 0, 0)),
        out_shape=jax.ShapeDtypeStruct((_B, Sp, D), jnp.float32),
        scratch_shapes=[pltpu.VMEM((Sp, D), jnp.float32)] * 4,
        compiler_params=pltpu.CompilerParams(
            dimension_semantics=("arbitrary", "arbitrary")),
    )(starts, kv, s1p, wq, bq8, wo, bo8)


def _attn_block(x, kk, vv, qh_all, wo, bo8, n_seeds, D):
    hd = D // _HEADS
    scale = 1.0 / math.sqrt(D)
    Sp = kk.shape[0]
    cols = jax.lax.broadcasted_iota(jnp.int32, (1, Sp), 1)
    keymask = cols < n_seeds
    outs = []
    for h in range(_HEADS):
        lo, hi = h * hd, (h + 1) * hd
        qh = qh_all[:, lo:hi]
        kh = kk[:, lo:hi]
        vh = vv[:, lo:hi]
        s = jax.lax.dot_general(qh.astype(jnp.bfloat16), kh.astype(jnp.bfloat16),
                                (((1,), (1,)), ((), ())),
                                preferred_element_type=jnp.float32) * scale
        s = jnp.where(keymask, s, -1e30)
        m = jnp.max(s, axis=1, keepdims=True)
        p = jnp.exp(s - m)
        l = jnp.sum(p, axis=1, keepdims=True)
        a = p * (1.0 / l)
        outs.append(qh + jnp.dot(a.astype(jnp.bfloat16), vh.astype(jnp.bfloat16),
                                 preferred_element_type=jnp.float32))
    o = jnp.concatenate(outs, axis=1)
    o2 = jnp.dot(o, wo, preferred_element_type=jnp.float32)
    return o + jnp.maximum(o2 + bo8[0:1, :], 0.0)


def _gmt_tail(bx1, n_seeds, sab, pma2, s2p, lin2w, lin2b8, outw, outb8):
    Bt, Sp, D = bx1.shape

    (swq, sbq8, swk, sbk8, swv, sbv8, swo, sbo8) = sab
    (pwq, pbq8, pwk, pbk8, pwv, pbv8, pwo, pbo8) = pma2

    def kern(x_ref, swq_r, sbq_r, swk_r, sbk_r, swv_r, sbv_r, swo_r, sbo_r,
             s2_r, pwq_r, pbq_r, pwk_r, pbk_r, pwv_r, pbv_r, pwo_r, pbo_r,
             l2w_r, l2b_r, ow_r, ob_r, o_ref):
        x = x_ref[0]
        qp = jnp.dot(x, swq_r[...], preferred_element_type=jnp.float32) + sbq_r[0:1, :]
        kk = jnp.dot(x, swk_r[...], preferred_element_type=jnp.float32) + sbk_r[0:1, :]
        vv = jnp.dot(x, swv_r[...], preferred_element_type=jnp.float32) + sbv_r[0:1, :]
        x2 = _attn_block(x, kk, vv, qp, swo_r[...], sbo_r[...], n_seeds, D)
        q2 = jnp.dot(s2_r[...], pwq_r[...], preferred_element_type=jnp.float32) + pbq_r[0:1, :]
        k2 = jnp.dot(x2, pwk_r[...], preferred_element_type=jnp.float32) + pbk_r[0:1, :]
        v2 = jnp.dot(x2, pwv_r[...], preferred_element_type=jnp.float32) + pbv_r[0:1, :]
        x3 = _attn_block(None, k2, v2, q2, pwo_r[...], pbo_r[...], n_seeds, D)
        gx = jnp.dot(x3, l2w_r[...], preferred_element_type=jnp.float32) + l2b_r[0:1, :]
        y = jnp.dot(gx, ow_r[...], preferred_element_type=jnp.float32) + ob_r[0:1, :]
        o_ref[0] = y

    wb = lambda shape: pl.BlockSpec(shape, lambda b: (0, 0))
    return pl.pallas_call(
        kern,
        grid=(Bt,),
        in_specs=[pl.BlockSpec((1, Sp, D), lambda b: (b, 0, 0))]
        + [wb((D, D)), wb((8, D))] * 4
        + [wb((8, D))]
        + [wb((D, D)), wb((8, D))] * 4
        + [wb((D, D)), wb((8, D)), wb((D, D)), wb((8, D))],
        out_specs=pl.BlockSpec((1, 8, D), lambda b: (b, 0, 0)),
        out_shape=jax.ShapeDtypeStruct((Bt, 8, D), jnp.float32),
        compiler_params=pltpu.CompilerParams(
            dimension_semantics=("arbitrary",)),
    )(bx1, swq, sbq8, swk, sbk8, swv, sbv8, swo, sbo8,
      s2p, pwq, pbq8, pwk, pbk8, pwv, pbv8, pwo, pbo8,
      lin2w, lin2b8, outw, outb8)


def _row8(b):
    return jnp.broadcast_to(b[None, :], (8, b.shape[0])).astype(jnp.float32)


def kernel(x, edge_index, edge_weight, batch, params):
    del edge_weight
    N, D = x.shape
    Np = _rup(N, 2048)
    bf = jnp.bfloat16
    src, dst = edge_index[0], edge_index[1]

    flat = dst.astype(jnp.int32) * Np + src.astype(jnp.int32)
    A = jnp.zeros((Np * Np,), jnp.float32).at[flat].add(1.0)
    A = A.reshape(Np, Np).astype(bf)

    h_bf = jnp.pad(x, ((0, Np - N), (0, 0))).astype(bf)
    add_prev = jnp.pad(x, ((0, Np - N), (0, 0)))
    indeg = None
    for lyr in params["gin"]:
        w1, b1 = lyr["lin1"]["W"], lyr["lin1"]["b"]
        w2, b2 = lyr["lin2"]["W"], lyr["lin2"]["b"]
        if indeg is None:
            rhs = jnp.concatenate([h_bf, jnp.ones((Np, 128), bf)], axis=1)
            ad = jnp.concatenate(
                [add_prev, jnp.zeros((Np, 128), add_prev.dtype)], axis=1)
            both = _mm(A, rhs, addend=ad, out_dtype=bf)
            h2in = both[:, :h_bf.shape[1]]
            indeg = both[:, h_bf.shape[1]:]
        else:
            h2in = _mm(A, h_bf, addend=add_prev, out_dtype=bf)
        t = _mm(h2in, w1.astype(bf), bias=_row8(b1), relu=True, out_dtype=bf)
        z = _mm(t, w2.astype(bf), bias=_row8(b2))
        stats = _colstats(z, N)
        h_bf = _bn_relu(z, stats, _row8(lyr["bn_g"]), _row8(lyr["bn_b"]), N)
        add_prev = h_bf

    h128 = _mm(h_bf, params["proj"]["W"].astype(bf),
               bias=_row8(params["proj"]["b"]), out_dtype=bf)
    g = params["gmt"]
    hx_bf = _mm(h128, g["lin1"]["W"].astype(bf),
                bias=_row8(g["lin1"]["b"]), out_dtype=bf)

    p1 = g["pma1"]
    wkv = jnp.concatenate([p1["gcn_k"]["W"], p1["gcn_v"]["W"]], axis=1)
    bkv = jnp.concatenate([p1["gcn_k"]["b"], p1["gcn_v"]["b"]])
    hkv = _mm(hx_bf, wkv.astype(bf))
    u = _scale_dinv(hkv, indeg, N, out_dtype=bf)
    w_agg = _mm(A, u, addend=u)
    kv = _scale_dinv(w_agg, indeg, N, bias8=_row8(bkv))

    starts = jnp.searchsorted(
        batch, jnp.arange(_B + 1, dtype=batch.dtype), side="left").astype(jnp.int32)
    starts = jnp.pad(starts, (0, 72 - (_B + 1)))

    S1 = p1["S"][0]
    n_seeds = S1.shape[0]
    Sp = _rup(n_seeds, 8)
    s1p = jnp.pad(S1, ((0, Sp - n_seeds), (0, 0)))
    bx1 = _pma1(kv, s1p, p1["fc_q"]["W"], _row8(p1["fc_q"]["b"]),
                p1["fc_o"]["W"], _row8(p1["fc_o"]["b"]), starts, n_seeds)

    s = g["sab"]
    p2 = g["pma2"]
    s2p = jnp.pad(p2["S"][0], ((0, 8 - p2["S"].shape[1]), (0, 0)))
    outw = jnp.pad(params["out"]["W"], ((0, 0), (0, D - params["out"]["W"].shape[1])))
    outb = jnp.pad(params["out"]["b"], (0, D - params["out"]["b"].shape[0]))
    y = _gmt_tail(
        bx1, n_seeds,
        (s["fc_q"]["W"], _row8(s["fc_q"]["b"]),
         s["layer_k"]["W"], _row8(s["layer_k"]["b"]),
         s["layer_v"]["W"], _row8(s["layer_v"]["b"]),
         s["fc_o"]["W"], _row8(s["fc_o"]["b"])),
        (p2["fc_q"]["W"], _row8(p2["fc_q"]["b"]),
         p2["layer_k"]["W"], _row8(p2["layer_k"]["b"]),
         p2["layer_v"]["W"], _row8(p2["layer_v"]["b"]),
         p2["fc_o"]["W"], _row8(p2["fc_o"]["b"])),
        s2p, g["lin2"]["W"], _row8(g["lin2"]["b"]), outw, _row8(outb))
    return y[:, 0, :params["out"]["W"].shape[1]]


# speedup vs baseline: 4.7571x; 1.0093x over previous
import math

import jax
import jax.numpy as jnp
from jax.experimental import pallas as pl
from jax.experimental.pallas import tpu as pltpu

_B = 64
_HEADS = 4


def _cdiv(a, b):
    return (a + b - 1) // b


def _rup(a, b):
    return _cdiv(a, b) * b


def _mm(x, y, *, bias=None, addend=None, relu=False, out_dtype=jnp.float32,
        stats_n=None):
    M, K = x.shape
    _, Nn = y.shape
    bm = 512 if M % 512 == 0 else min(256, M)
    bn = min(1024, Nn)
    bk = min(512, K)
    gm, gn, gk = M // bm, Nn // bn, K // bk

    def kern(*refs):
        i = 0
        x_ref, y_ref = refs[0], refs[1]
        nxt = 2
        b_ref = ad_ref = None
        if bias is not None:
            b_ref = refs[nxt]
            nxt += 1
        if addend is not None:
            ad_ref = refs[nxt]
            nxt += 1
        if stats_n is not None:
            o_ref, s_ref, acc_ref = refs[nxt], refs[nxt + 1], refs[nxt + 2]
        else:
            o_ref, acc_ref = refs[nxt], refs[nxt + 1]
        i = pl.program_id(0)
        k = pl.program_id(2)

        @pl.when(k == 0)
        def _():
            acc_ref[...] = jnp.zeros_like(acc_ref)

        acc_ref[...] += jnp.dot(x_ref[...], y_ref[...],
                                preferred_element_type=jnp.float32)

        @pl.when(k == gk - 1)
        def _():
            r = acc_ref[...]
            if b_ref is not None:
                r = r + b_ref[0:1, :]
            if ad_ref is not None:
                r = r + ad_ref[...].astype(jnp.float32)
            if relu:
                r = jnp.maximum(r, 0.0)
            o_ref[...] = r.astype(out_dtype)
            if stats_n is not None:
                @pl.when(i == 0)
                def _():
                    s_ref[...] = jnp.zeros_like(s_ref)

                rows = jax.lax.broadcasted_iota(jnp.int32, (bm, 1), 0) + i * bm
                rm = jnp.where(rows < stats_n, r, 0.0)
                s_ref[0:1, :] += jnp.sum(rm, axis=0, keepdims=True)
                s_ref[1:2, :] += jnp.sum(rm * rm, axis=0, keepdims=True)

    in_specs = [
        pl.BlockSpec((bm, bk), lambda i, j, k: (i, k)),
        pl.BlockSpec((bk, bn), lambda i, j, k: (k, j)),
    ]
    ops = [x, y]
    if bias is not None:
        in_specs.append(pl.BlockSpec((8, bn), lambda i, j, k: (0, j)))
        ops.append(bias)
    if addend is not None:
        in_specs.append(pl.BlockSpec((bm, bn), lambda i, j, k: (i, j)))
        ops.append(addend)
    out_specs = pl.BlockSpec((bm, bn), lambda i, j, k: (i, j))
    out_shape = jax.ShapeDtypeStruct((M, Nn), out_dtype)
    sem = ("parallel", "parallel", "arbitrary")
    if stats_n is not None:
        out_specs = [out_specs, pl.BlockSpec((8, bn), lambda i, j, k: (0, j))]
        out_shape = [out_shape, jax.ShapeDtypeStruct((8, Nn), jnp.float32)]
        sem = ("arbitrary", "parallel", "arbitrary")
    return pl.pallas_call(
        kern,
        grid=(gm, gn, gk),
        in_specs=in_specs,
        out_specs=out_specs,
        out_shape=out_shape,
        scratch_shapes=[pltpu.VMEM((bm, bn), jnp.float32)],
        compiler_params=pltpu.CompilerParams(dimension_semantics=sem),
    )(*ops)


def _colstats(z, n_valid):
    M, Dh = z.shape
    bm = min(512, M)
    gm = M // bm

    def kern(z_ref, o_ref):
        i = pl.program_id(0)

        @pl.when(i == 0)
        def _():
            o_ref[...] = jnp.zeros_like(o_ref)

        rows = jax.lax.broadcasted_iota(jnp.int32, (bm, 1), 0) + i * bm
        zz = jnp.where(rows < n_valid, z_ref[...], 0.0)
        o_ref[0:1, :] += jnp.sum(zz, axis=0, keepdims=True)
        o_ref[1:2, :] += jnp.sum(zz * zz, axis=0, keepdims=True)

    return pl.pallas_call(
        kern,
        grid=(gm,),
        in_specs=[pl.BlockSpec((bm, Dh), lambda i: (i, 0))],
        out_specs=pl.BlockSpec((8, Dh), lambda i: (0, 0)),
        out_shape=jax.ShapeDtypeStruct((8, Dh), jnp.float32),
        compiler_params=pltpu.CompilerParams(
            dimension_semantics=("arbitrary",)),
    )(z)


def _bn_relu(z, stats, g8, b8, n_valid):
    M, Dh = z.shape
    bm = min(512, M)

    def kern(z_ref, s_ref, g_ref, b_ref, o_ref):
        inv_n = 1.0 / n_valid
        mu = s_ref[0:1, :] * inv_n
        var = s_ref[1:2, :] * inv_n - mu * mu
        scale = jax.lax.rsqrt(var + 1e-5) * g_ref[0:1, :]
        r = (z_ref[...] - mu) * scale + b_ref[0:1, :]
        o_ref[...] = jnp.maximum(r, 0.0).astype(jnp.bfloat16)

    return pl.pallas_call(
        kern,
        grid=(M // bm,),
        in_specs=[
            pl.BlockSpec((bm, Dh), lambda i: (i, 0)),
            pl.BlockSpec((8, Dh), lambda i: (0, 0)),
            pl.BlockSpec((8, Dh), lambda i: (0, 0)),
            pl.BlockSpec((8, Dh), lambda i: (0, 0)),
        ],
        out_specs=pl.BlockSpec((bm, Dh), lambda i: (i, 0)),
        out_shape=jax.ShapeDtypeStruct((M, Dh), jnp.bfloat16),
        compiler_params=pltpu.CompilerParams(
            dimension_semantics=("arbitrary",)),
    )(z, stats, g8, b8)


def _scale_dinv(v, indeg, n_valid, bias8=None, out_dtype=jnp.float32):
    M, C = v.shape
    bm = min(512, M)

    def kern(*refs):
        if bias8 is not None:
            v_ref, ind_ref, b_ref, o_ref = refs
        else:
            v_ref, ind_ref, o_ref = refs
            b_ref = None
        i = pl.program_id(0)
        rows = jax.lax.broadcasted_iota(jnp.int32, (bm, 1), 0) + i * bm
        ind = ind_ref[...][:, 0:1].astype(jnp.float32)
        dinv = jnp.where(rows < n_valid, jax.lax.rsqrt(ind + 1.0), 0.0)
        r = v_ref[...].astype(jnp.float32) * dinv
        if b_ref is not None:
            r = r + b_ref[0:1, :]
        o_ref[...] = r.astype(out_dtype)

    in_specs = [
        pl.BlockSpec((bm, C), lambda i: (i, 0)),
        pl.BlockSpec((bm, 128), lambda i: (i, 0)),
    ]
    ops = [v, indeg]
    if bias8 is not None:
        in_specs.append(pl.BlockSpec((8, C), lambda i: (0, 0)))
        ops.append(bias8)
    return pl.pallas_call(
        kern,
        grid=(M // bm,),
        in_specs=in_specs,
        out_specs=pl.BlockSpec((bm, C), lambda i: (i, 0)),
        out_shape=jax.ShapeDtypeStruct((M, C), out_dtype),
        compiler_params=pltpu.CompilerParams(
            dimension_semantics=("arbitrary",)),
    )(*ops)


def _pma1(kv, s1p, wq, bq8, wo, bo8, starts, n_seeds):
    Np, C = kv.shape
    D = C // 2
    hd = D // _HEADS
    Sp, _ = s1p.shape
    T = 1024 if Np % 1024 == 0 else 512
    NT = Np // T
    scale = 1.0 / math.sqrt(D)

    def kern(st_ref, kv_ref, s1_ref, wq_ref, bq_ref, wo_ref, bo_ref,
             o_ref, q_ref, m_ref, l_ref, acc_ref):
        b = pl.program_id(0)
        t = pl.program_id(1)
        start = st_ref[b]
        end = st_ref[b + 1]

        @pl.when(t == 0)
        def _():
            q_ref[...] = jnp.dot(s1_ref[...], wq_ref[...],
                                 preferred_element_type=jnp.float32) + bq_ref[0:1, :]
            m_ref[...] = jnp.full((Sp, D), -1e30, jnp.float32)
            l_ref[...] = jnp.zeros((Sp, D), jnp.float32)
            acc_ref[...] = jnp.zeros((Sp, D), jnp.float32)

        kt = kv_ref[pl.ds(t * T, T), :]
        ids = jax.lax.broadcasted_iota(jnp.int32, (1, T), 1) + t * T
        valid = (ids >= start) & (ids < end)
        qp = q_ref[...]
        for h in range(_HEADS):
            lo, hi = h * hd, (h + 1) * hd
            qh = qp[:, lo:hi].astype(jnp.bfloat16)
            kh = kt[:, lo:hi].astype(jnp.bfloat16)
            vh = kt[:, D + lo:D + hi].astype(jnp.bfloat16)
            s = jax.lax.dot_general(qh, kh, (((1,), (1,)), ((), ())),
                                    preferred_element_type=jnp.float32) * scale
            s = jnp.where(valid, s, -1e30)
            mold = m_ref[:, lo:hi][:, 0:1]
            mnew = jnp.maximum(mold, jnp.max(s, axis=1, keepdims=True))
            p = jnp.where(valid, jnp.exp(s - mnew), 0.0)
            resc = jnp.exp(mold - mnew)
            lnew = l_ref[:, lo:hi][:, 0:1] * resc + jnp.sum(p, axis=1, keepdims=True)
            accn = acc_ref[:, lo:hi] * resc + jnp.dot(
                p.astype(jnp.bfloat16), vh, preferred_element_type=jnp.float32)
            m_ref[:, lo:hi] = jnp.broadcast_to(mnew, (Sp, hd))
            l_ref[:, lo:hi] = jnp.broadcast_to(lnew, (Sp, hd))
            acc_ref[:, lo:hi] = accn

        @pl.when(t == NT - 1)
        def _():
            l = l_ref[...]
            attn = acc_ref[...] * jnp.where(l > 0, 1.0 / l, 0.0)
            o = q_ref[...] + attn
            o2 = jnp.dot(o, wo_ref[...], preferred_element_type=jnp.float32)
            o_ref[0] = o + jnp.maximum(o2 + bo_ref[0:1, :], 0.0)

    return pl.pallas_call(
        kern,
        grid=(_B, NT),
        in_specs=[
            pl.BlockSpec(memory_space=pltpu.SMEM),
            pl.BlockSpec((Np, C), lambda b, t: (0, 0)),
            pl.BlockSpec((Sp, D), lambda b, t: (0, 0)),
            pl.BlockSpec((D, D), lambda b, t: (0, 0)),
            pl.BlockSpec((8, D), lambda b, t: (0, 0)),
            pl.BlockSpec((D, D), lambda b, t: (0, 0)),
            pl.BlockSpec((8, D), lambda b, t: (0, 0)),
        ],
        out_specs=pl.BlockSpec((1, Sp, D), lambda b, t: (b, 0, 0)),
        out_shape=jax.ShapeDtypeStruct((_B, Sp, D), jnp.float32),
        scratch_shapes=[pltpu.VMEM((Sp, D), jnp.float32)] * 4,
        compiler_params=pltpu.CompilerParams(
            dimension_semantics=("arbitrary", "arbitrary")),
    )(starts, kv, s1p, wq, bq8, wo, bo8)


def _attn_block(x, kk, vv, qh_all, wo, bo8, n_seeds, D):
    hd = D // _HEADS
    scale = 1.0 / math.sqrt(D)
    Sp = kk.shape[0]
    cols = jax.lax.broadcasted_iota(jnp.int32, (1, Sp), 1)
    keymask = cols < n_seeds
    outs = []
    for h in range(_HEADS):
        lo, hi = h * hd, (h + 1) * hd
        qh = qh_all[:, lo:hi]
        kh = kk[:, lo:hi]
        vh = vv[:, lo:hi]
        s = jax.lax.dot_general(qh.astype(jnp.bfloat16), kh.astype(jnp.bfloat16),
                                (((1,), (1,)), ((), ())),
                                preferred_element_type=jnp.float32) * scale
        s = jnp.where(keymask, s, -1e30)
        m = jnp.max(s, axis=1, keepdims=True)
        p = jnp.exp(s - m)
        l = jnp.sum(p, axis=1, keepdims=True)
        a = p * (1.0 / l)
        outs.append(qh + jnp.dot(a.astype(jnp.bfloat16), vh.astype(jnp.bfloat16),
                                 preferred_element_type=jnp.float32))
    o = jnp.concatenate(outs, axis=1)
    o2 = jnp.dot(o, wo, preferred_element_type=jnp.float32)
    return o + jnp.maximum(o2 + bo8[0:1, :], 0.0)


def _gmt_tail(bx1, n_seeds, sab, pma2, s2p, lin2w, lin2b8, outw, outb8):
    Bt, Sp, D = bx1.shape

    (swq, sbq8, swk, sbk8, swv, sbv8, swo, sbo8) = sab
    (pwq, pbq8, pwk, pbk8, pwv, pbv8, pwo, pbo8) = pma2

    def kern(x_ref, swq_r, sbq_r, swk_r, sbk_r, swv_r, sbv_r, swo_r, sbo_r,
             s2_r, pwq_r, pbq_r, pwk_r, pbk_r, pwv_r, pbv_r, pwo_r, pbo_r,
             l2w_r, l2b_r, ow_r, ob_r, o_ref):
        x = x_ref[0]
        qp = jnp.dot(x, swq_r[...], preferred_element_type=jnp.float32) + sbq_r[0:1, :]
        kk = jnp.dot(x, swk_r[...], preferred_element_type=jnp.float32) + sbk_r[0:1, :]
        vv = jnp.dot(x, swv_r[...], preferred_element_type=jnp.float32) + sbv_r[0:1, :]
        x2 = _attn_block(x, kk, vv, qp, swo_r[...], sbo_r[...], n_seeds, D)
        q2 = jnp.dot(s2_r[...], pwq_r[...], preferred_element_type=jnp.float32) + pbq_r[0:1, :]
        k2 = jnp.dot(x2, pwk_r[...], preferred_element_type=jnp.float32) + pbk_r[0:1, :]
        v2 = jnp.dot(x2, pwv_r[...], preferred_element_type=jnp.float32) + pbv_r[0:1, :]
        x3 = _attn_block(None, k2, v2, q2, pwo_r[...], pbo_r[...], n_seeds, D)
        gx = jnp.dot(x3, l2w_r[...], preferred_element_type=jnp.float32) + l2b_r[0:1, :]
        y = jnp.dot(gx, ow_r[...], preferred_element_type=jnp.float32) + ob_r[0:1, :]
        o_ref[0] = y

    wb = lambda shape: pl.BlockSpec(shape, lambda b: (0, 0))
    return pl.pallas_call(
        kern,
        grid=(Bt,),
        in_specs=[pl.BlockSpec((1, Sp, D), lambda b: (b, 0, 0))]
        + [wb((D, D)), wb((8, D))] * 4
        + [wb((8, D))]
        + [wb((D, D)), wb((8, D))] * 4
        + [wb((D, D)), wb((8, D)), wb((D, D)), wb((8, D))],
        out_specs=pl.BlockSpec((1, 8, D), lambda b: (b, 0, 0)),
        out_shape=jax.ShapeDtypeStruct((Bt, 8, D), jnp.float32),
        compiler_params=pltpu.CompilerParams(
            dimension_semantics=("arbitrary",)),
    )(bx1, swq, sbq8, swk, sbk8, swv, sbv8, swo, sbo8,
      s2p, pwq, pbq8, pwk, pbk8, pwv, pbv8, pwo, pbo8,
      lin2w, lin2b8, outw, outb8)


def _row8(b):
    return jnp.broadcast_to(b[None, :], (8, b.shape[0])).astype(jnp.float32)


def kernel(x, edge_index, edge_weight, batch, params):
    del edge_weight
    N, D = x.shape
    Np = _rup(N, 2048)
    bf = jnp.bfloat16
    src, dst = edge_index[0], edge_index[1]

    flat = dst.astype(jnp.int32) * Np + src.astype(jnp.int32)
    A = jnp.zeros((Np * Np,), jnp.int32).at[flat].add(1)
    A = A.reshape(Np, Np).astype(bf)

    h_bf = jnp.pad(x, ((0, Np - N), (0, 0))).astype(bf)
    add_prev = jnp.pad(x, ((0, Np - N), (0, 0)))
    indeg = None
    for lyr in params["gin"]:
        w1, b1 = lyr["lin1"]["W"], lyr["lin1"]["b"]
        w2, b2 = lyr["lin2"]["W"], lyr["lin2"]["b"]
        if indeg is None:
            rhs = jnp.concatenate([h_bf, jnp.ones((Np, 128), bf)], axis=1)
            ad = jnp.concatenate(
                [add_prev, jnp.zeros((Np, 128), add_prev.dtype)], axis=1)
            both = _mm(A, rhs, addend=ad, out_dtype=bf)
            h2in = both[:, :h_bf.shape[1]]
            indeg = both[:, h_bf.shape[1]:]
        else:
            h2in = _mm(A, h_bf, addend=add_prev, out_dtype=bf)
        t = _mm(h2in, w1.astype(bf), bias=_row8(b1), relu=True, out_dtype=bf)
        z, stats = _mm(t, w2.astype(bf), bias=_row8(b2), stats_n=N)
        h_bf = _bn_relu(z, stats, _row8(lyr["bn_g"]), _row8(lyr["bn_b"]), N)
        add_prev = h_bf

    h128 = _mm(h_bf, params["proj"]["W"].astype(bf),
               bias=_row8(params["proj"]["b"]), out_dtype=bf)
    g = params["gmt"]
    hx_bf = _mm(h128, g["lin1"]["W"].astype(bf),
                bias=_row8(g["lin1"]["b"]), out_dtype=bf)

    p1 = g["pma1"]
    wkv = jnp.concatenate([p1["gcn_k"]["W"], p1["gcn_v"]["W"]], axis=1)
    bkv = jnp.concatenate([p1["gcn_k"]["b"], p1["gcn_v"]["b"]])
    hkv = _mm(hx_bf, wkv.astype(bf))
    u = _scale_dinv(hkv, indeg, N, out_dtype=bf)
    w_agg = _mm(A, u, addend=u)
    kv = _scale_dinv(w_agg, indeg, N, bias8=_row8(bkv))

    starts = jnp.searchsorted(
        batch, jnp.arange(_B + 1, dtype=batch.dtype), side="left").astype(jnp.int32)
    starts = jnp.pad(starts, (0, 72 - (_B + 1)))

    S1 = p1["S"][0]
    n_seeds = S1.shape[0]
    Sp = _rup(n_seeds, 8)
    s1p = jnp.pad(S1, ((0, Sp - n_seeds), (0, 0)))
    bx1 = _pma1(kv, s1p, p1["fc_q"]["W"], _row8(p1["fc_q"]["b"]),
                p1["fc_o"]["W"], _row8(p1["fc_o"]["b"]), starts, n_seeds)

    s = g["sab"]
    p2 = g["pma2"]
    s2p = jnp.pad(p2["S"][0], ((0, 8 - p2["S"].shape[1]), (0, 0)))
    outw = jnp.pad(params["out"]["W"], ((0, 0), (0, D - params["out"]["W"].shape[1])))
    outb = jnp.pad(params["out"]["b"], (0, D - params["out"]["b"].shape[0]))
    y = _gmt_tail(
        bx1, n_seeds,
        (s["fc_q"]["W"], _row8(s["fc_q"]["b"]),
         s["layer_k"]["W"], _row8(s["layer_k"]["b"]),
         s["layer_v"]["W"], _row8(s["layer_v"]["b"]),
         s["fc_o"]["W"], _row8(s["fc_o"]["b"])),
        (p2["fc_q"]["W"], _row8(p2["fc_q"]["b"]),
         p2["layer_k"]["W"], _row8(p2["layer_k"]["b"]),
         p2["layer_v"]["W"], _row8(p2["layer_v"]["b"]),
         p2["fc_o"]["W"], _row8(p2["fc_o"]["b"])),
        s2p, g["lin2"]["W"], _row8(g["lin2"]["b"]), outw, _row8(outb))
    return y[:, 0, :params["out"]["W"].shape[1]]


# speedup vs baseline: 6.0659x; 1.2751x over previous
import math

import jax
import jax.numpy as jnp
from jax.experimental import pallas as pl
from jax.experimental.pallas import tpu as pltpu

_B = 64
_HEADS = 4


def _cdiv(a, b):
    return (a + b - 1) // b


def _rup(a, b):
    return _cdiv(a, b) * b


def _mm(x, y, *, bias=None, addend=None, relu=False, out_dtype=jnp.float32,
        stats_n=None):
    M, K = x.shape
    _, Nn = y.shape
    bm = 512 if M % 512 == 0 else min(256, M)
    bn = min(1024, Nn)
    bk = min(512, K)
    gm, gn, gk = M // bm, Nn // bn, K // bk

    def kern(*refs):
        i = 0
        x_ref, y_ref = refs[0], refs[1]
        nxt = 2
        b_ref = ad_ref = None
        if bias is not None:
            b_ref = refs[nxt]
            nxt += 1
        if addend is not None:
            ad_ref = refs[nxt]
            nxt += 1
        if stats_n is not None:
            o_ref, s_ref, acc_ref = refs[nxt], refs[nxt + 1], refs[nxt + 2]
        else:
            o_ref, acc_ref = refs[nxt], refs[nxt + 1]
        i = pl.program_id(0)
        k = pl.program_id(2)

        @pl.when(k == 0)
        def _():
            acc_ref[...] = jnp.zeros_like(acc_ref)

        acc_ref[...] += jnp.dot(x_ref[...], y_ref[...],
                                preferred_element_type=jnp.float32)

        @pl.when(k == gk - 1)
        def _():
            r = acc_ref[...]
            if b_ref is not None:
                r = r + b_ref[0:1, :]
            if ad_ref is not None:
                r = r + ad_ref[...].astype(jnp.float32)
            if relu:
                r = jnp.maximum(r, 0.0)
            o_ref[...] = r.astype(out_dtype)
            if stats_n is not None:
                @pl.when(i == 0)
                def _():
                    s_ref[...] = jnp.zeros_like(s_ref)

                rows = jax.lax.broadcasted_iota(jnp.int32, (bm, 1), 0) + i * bm
                rm = jnp.where(rows < stats_n, r, 0.0)
                s_ref[0:1, :] += jnp.sum(rm, axis=0, keepdims=True)
                s_ref[1:2, :] += jnp.sum(rm * rm, axis=0, keepdims=True)

    in_specs = [
        pl.BlockSpec((bm, bk), lambda i, j, k: (i, k)),
        pl.BlockSpec((bk, bn), lambda i, j, k: (k, j)),
    ]
    ops = [x, y]
    if bias is not None:
        in_specs.append(pl.BlockSpec((8, bn), lambda i, j, k: (0, j)))
        ops.append(bias)
    if addend is not None:
        in_specs.append(pl.BlockSpec((bm, bn), lambda i, j, k: (i, j)))
        ops.append(addend)
    out_specs = pl.BlockSpec((bm, bn), lambda i, j, k: (i, j))
    out_shape = jax.ShapeDtypeStruct((M, Nn), out_dtype)
    sem = ("parallel", "parallel", "arbitrary")
    if stats_n is not None:
        out_specs = [out_specs, pl.BlockSpec((8, bn), lambda i, j, k: (0, j))]
        out_shape = [out_shape, jax.ShapeDtypeStruct((8, Nn), jnp.float32)]
        sem = ("arbitrary", "parallel", "arbitrary")
    return pl.pallas_call(
        kern,
        grid=(gm, gn, gk),
        in_specs=in_specs,
        out_specs=out_specs,
        out_shape=out_shape,
        scratch_shapes=[pltpu.VMEM((bm, bn), jnp.float32)],
        compiler_params=pltpu.CompilerParams(dimension_semantics=sem),
    )(*ops)


def _colstats(z, n_valid):
    M, Dh = z.shape
    bm = min(512, M)
    gm = M // bm

    def kern(z_ref, o_ref):
        i = pl.program_id(0)

        @pl.when(i == 0)
        def _():
            o_ref[...] = jnp.zeros_like(o_ref)

        rows = jax.lax.broadcasted_iota(jnp.int32, (bm, 1), 0) + i * bm
        zz = jnp.where(rows < n_valid, z_ref[...], 0.0)
        o_ref[0:1, :] += jnp.sum(zz, axis=0, keepdims=True)
        o_ref[1:2, :] += jnp.sum(zz * zz, axis=0, keepdims=True)

    return pl.pallas_call(
        kern,
        grid=(gm,),
        in_specs=[pl.BlockSpec((bm, Dh), lambda i: (i, 0))],
        out_specs=pl.BlockSpec((8, Dh), lambda i: (0, 0)),
        out_shape=jax.ShapeDtypeStruct((8, Dh), jnp.float32),
        compiler_params=pltpu.CompilerParams(
            dimension_semantics=("arbitrary",)),
    )(z)


def _bn_relu(z, stats, g8, b8, n_valid):
    M, Dh = z.shape
    bm = min(512, M)

    def kern(z_ref, s_ref, g_ref, b_ref, o_ref):
        inv_n = 1.0 / n_valid
        mu = s_ref[0:1, :] * inv_n
        var = s_ref[1:2, :] * inv_n - mu * mu
        scale = jax.lax.rsqrt(var + 1e-5) * g_ref[0:1, :]
        r = (z_ref[...] - mu) * scale + b_ref[0:1, :]
        o_ref[...] = jnp.maximum(r, 0.0).astype(jnp.bfloat16)

    return pl.pallas_call(
        kern,
        grid=(M // bm,),
        in_specs=[
            pl.BlockSpec((bm, Dh), lambda i: (i, 0)),
            pl.BlockSpec((8, Dh), lambda i: (0, 0)),
            pl.BlockSpec((8, Dh), lambda i: (0, 0)),
            pl.BlockSpec((8, Dh), lambda i: (0, 0)),
        ],
        out_specs=pl.BlockSpec((bm, Dh), lambda i: (i, 0)),
        out_shape=jax.ShapeDtypeStruct((M, Dh), jnp.bfloat16),
        compiler_params=pltpu.CompilerParams(
            dimension_semantics=("arbitrary",)),
    )(z, stats, g8, b8)


def _scale_dinv(v, indeg, n_valid, bias8=None, out_dtype=jnp.float32):
    M, C = v.shape
    bm = min(512, M)

    def kern(*refs):
        if bias8 is not None:
            v_ref, ind_ref, b_ref, o_ref = refs
        else:
            v_ref, ind_ref, o_ref = refs
            b_ref = None
        i = pl.program_id(0)
        rows = jax.lax.broadcasted_iota(jnp.int32, (bm, 1), 0) + i * bm
        ind = ind_ref[...][:, 0:1].astype(jnp.float32)
        dinv = jnp.where(rows < n_valid, jax.lax.rsqrt(ind + 1.0), 0.0)
        r = v_ref[...].astype(jnp.float32) * dinv
        if b_ref is not None:
            r = r + b_ref[0:1, :]
        o_ref[...] = r.astype(out_dtype)

    in_specs = [
        pl.BlockSpec((bm, C), lambda i: (i, 0)),
        pl.BlockSpec((bm, 128), lambda i: (i, 0)),
    ]
    ops = [v, indeg]
    if bias8 is not None:
        in_specs.append(pl.BlockSpec((8, C), lambda i: (0, 0)))
        ops.append(bias8)
    return pl.pallas_call(
        kern,
        grid=(M // bm,),
        in_specs=in_specs,
        out_specs=pl.BlockSpec((bm, C), lambda i: (i, 0)),
        out_shape=jax.ShapeDtypeStruct((M, C), out_dtype),
        compiler_params=pltpu.CompilerParams(
            dimension_semantics=("arbitrary",)),
    )(*ops)


def _pma1(kv, s1p, wq, bq8, wo, bo8, starts, n_seeds):
    Np, C = kv.shape
    D = C // 2
    hd = D // _HEADS
    Sp, _ = s1p.shape
    T = 1024 if Np % 1024 == 0 else 512
    NT = Np // T
    scale = 1.0 / math.sqrt(D)

    def kern(st_ref, kv_ref, s1_ref, wq_ref, bq_ref, wo_ref, bo_ref,
             o_ref, q_ref, m_ref, l_ref, acc_ref):
        b = pl.program_id(0)
        t = pl.program_id(1)
        start = st_ref[b]
        end = st_ref[b + 1]

        @pl.when(t == 0)
        def _():
            q_ref[...] = jnp.dot(s1_ref[...], wq_ref[...],
                                 preferred_element_type=jnp.float32) + bq_ref[0:1, :]
            m_ref[...] = jnp.full((Sp, D), -1e30, jnp.float32)
            l_ref[...] = jnp.zeros((Sp, D), jnp.float32)
            acc_ref[...] = jnp.zeros((Sp, D), jnp.float32)

        @pl.when((t * T < end) & ((t + 1) * T > start))
        def _():
            kt = kv_ref[pl.ds(t * T, T), :]
            ids = jax.lax.broadcasted_iota(jnp.int32, (1, T), 1) + t * T
            valid = (ids >= start) & (ids < end)
            qp = q_ref[...]
            for h in range(_HEADS):
                lo, hi = h * hd, (h + 1) * hd
                qh = qp[:, lo:hi].astype(jnp.bfloat16)
                kh = kt[:, lo:hi].astype(jnp.bfloat16)
                vh = kt[:, D + lo:D + hi].astype(jnp.bfloat16)
                s = jax.lax.dot_general(qh, kh, (((1,), (1,)), ((), ())),
                                        preferred_element_type=jnp.float32) * scale
                s = jnp.where(valid, s, -1e30)
                mold = m_ref[:, lo:hi][:, 0:1]
                mnew = jnp.maximum(mold, jnp.max(s, axis=1, keepdims=True))
                p = jnp.where(valid, jnp.exp(s - mnew), 0.0)
                resc = jnp.exp(mold - mnew)
                lnew = l_ref[:, lo:hi][:, 0:1] * resc + jnp.sum(p, axis=1, keepdims=True)
                accn = acc_ref[:, lo:hi] * resc + jnp.dot(
                    p.astype(jnp.bfloat16), vh, preferred_element_type=jnp.float32)
                m_ref[:, lo:hi] = jnp.broadcast_to(mnew, (Sp, hd))
                l_ref[:, lo:hi] = jnp.broadcast_to(lnew, (Sp, hd))
                acc_ref[:, lo:hi] = accn

        @pl.when(t == NT - 1)
        def _():
            l = l_ref[...]
            attn = acc_ref[...] * jnp.where(l > 0, 1.0 / l, 0.0)
            o = q_ref[...] + attn
            o2 = jnp.dot(o, wo_ref[...], preferred_element_type=jnp.float32)
            o_ref[0] = o + jnp.maximum(o2 + bo_ref[0:1, :], 0.0)

    return pl.pallas_call(
        kern,
        grid=(_B, NT),
        in_specs=[
            pl.BlockSpec(memory_space=pltpu.SMEM),
            pl.BlockSpec((Np, C), lambda b, t: (0, 0)),
            pl.BlockSpec((Sp, D), lambda b, t: (0, 0)),
            pl.BlockSpec((D, D), lambda b, t: (0, 0)),
            pl.BlockSpec((8, D), lambda b, t: (0, 0)),
            pl.BlockSpec((D, D), lambda b, t: (0, 0)),
            pl.BlockSpec((8, D), lambda b, t: (0, 0)),
        ],
        out_specs=pl.BlockSpec((1, Sp, D), lambda b, t: (b, 0, 0)),
        out_shape=jax.ShapeDtypeStruct((_B, Sp, D), jnp.float32),
        scratch_shapes=[pltpu.VMEM((Sp, D), jnp.float32)] * 4,
        compiler_params=pltpu.CompilerParams(
            dimension_semantics=("arbitrary", "arbitrary")),
    )(starts, kv, s1p, wq, bq8, wo, bo8)


def _attn_block(x, kk, vv, qh_all, wo, bo8, n_seeds, D):
    hd = D // _HEADS
    scale = 1.0 / math.sqrt(D)
    Sp = kk.shape[0]
    cols = jax.lax.broadcasted_iota(jnp.int32, (1, Sp), 1)
    keymask = cols < n_seeds
    outs = []
    for h in range(_HEADS):
        lo, hi = h * hd, (h + 1) * hd
        qh = qh_all[:, lo:hi]
        kh = kk[:, lo:hi]
        vh = vv[:, lo:hi]
        s = jax.lax.dot_general(qh.astype(jnp.bfloat16), kh.astype(jnp.bfloat16),
                                (((1,), (1,)), ((), ())),
                                preferred_element_type=jnp.float32) * scale
        s = jnp.where(keymask, s, -1e30)
        m = jnp.max(s, axis=1, keepdims=True)
        p = jnp.exp(s - m)
        l = jnp.sum(p, axis=1, keepdims=True)
        a = p * (1.0 / l)
        outs.append(qh + jnp.dot(a.astype(jnp.bfloat16), vh.astype(jnp.bfloat16),
                                 preferred_element_type=jnp.float32))
    o = jnp.concatenate(outs, axis=1)
    o2 = jnp.dot(o, wo, preferred_element_type=jnp.float32)
    return o + jnp.maximum(o2 + bo8[0:1, :], 0.0)


def _gmt_tail(bx1, n_seeds, sab, pma2, s2p, lin2w, lin2b8, outw, outb8):
    Bt, Sp, D = bx1.shape

    (swq, sbq8, swk, sbk8, swv, sbv8, swo, sbo8) = sab
    (pwq, pbq8, pwk, pbk8, pwv, pbv8, pwo, pbo8) = pma2

    def kern(x_ref, swq_r, sbq_r, swk_r, sbk_r, swv_r, sbv_r, swo_r, sbo_r,
             s2_r, pwq_r, pbq_r, pwk_r, pbk_r, pwv_r, pbv_r, pwo_r, pbo_r,
             l2w_r, l2b_r, ow_r, ob_r, o_ref):
        x = x_ref[0]
        qp = jnp.dot(x, swq_r[...], preferred_element_type=jnp.float32) + sbq_r[0:1, :]
        kk = jnp.dot(x, swk_r[...], preferred_element_type=jnp.float32) + sbk_r[0:1, :]
        vv = jnp.dot(x, swv_r[...], preferred_element_type=jnp.float32) + sbv_r[0:1, :]
        x2 = _attn_block(x, kk, vv, qp, swo_r[...], sbo_r[...], n_seeds, D)
        q2 = jnp.dot(s2_r[...], pwq_r[...], preferred_element_type=jnp.float32) + pbq_r[0:1, :]
        k2 = jnp.dot(x2, pwk_r[...], preferred_element_type=jnp.float32) + pbk_r[0:1, :]
        v2 = jnp.dot(x2, pwv_r[...], preferred_element_type=jnp.float32) + pbv_r[0:1, :]
        x3 = _attn_block(None, k2, v2, q2, pwo_r[...], pbo_r[...], n_seeds, D)
        gx = jnp.dot(x3, l2w_r[...], preferred_element_type=jnp.float32) + l2b_r[0:1, :]
        y = jnp.dot(gx, ow_r[...], preferred_element_type=jnp.float32) + ob_r[0:1, :]
        o_ref[0] = y

    wb = lambda shape: pl.BlockSpec(shape, lambda b: (0, 0))
    return pl.pallas_call(
        kern,
        grid=(Bt,),
        in_specs=[pl.BlockSpec((1, Sp, D), lambda b: (b, 0, 0))]
        + [wb((D, D)), wb((8, D))] * 4
        + [wb((8, D))]
        + [wb((D, D)), wb((8, D))] * 4
        + [wb((D, D)), wb((8, D)), wb((D, D)), wb((8, D))],
        out_specs=pl.BlockSpec((1, 8, D), lambda b: (b, 0, 0)),
        out_shape=jax.ShapeDtypeStruct((Bt, 8, D), jnp.float32),
        compiler_params=pltpu.CompilerParams(
            dimension_semantics=("arbitrary",)),
    )(bx1, swq, sbq8, swk, sbk8, swv, sbv8, swo, sbo8,
      s2p, pwq, pbq8, pwk, pbk8, pwv, pbv8, pwo, pbo8,
      lin2w, lin2b8, outw, outb8)


def _row8(b):
    return jnp.broadcast_to(b[None, :], (8, b.shape[0])).astype(jnp.float32)


def kernel(x, edge_index, edge_weight, batch, params):
    del edge_weight
    N, D = x.shape
    Np = _rup(N, 2048)
    bf = jnp.bfloat16
    src, dst = edge_index[0], edge_index[1]

    flat = dst.astype(jnp.int32) * Np + src.astype(jnp.int32)
    A = jnp.zeros((Np * Np,), jnp.int32).at[flat].add(1)
    A = A.reshape(Np, Np).astype(bf)

    h_bf = jnp.pad(x, ((0, Np - N), (0, 0))).astype(bf)
    add_prev = jnp.pad(x, ((0, Np - N), (0, 0)))
    indeg = None
    for lyr in params["gin"]:
        w1, b1 = lyr["lin1"]["W"], lyr["lin1"]["b"]
        w2, b2 = lyr["lin2"]["W"], lyr["lin2"]["b"]
        if indeg is None:
            rhs = jnp.concatenate([h_bf, jnp.ones((Np, 128), bf)], axis=1)
            ad = jnp.concatenate(
                [add_prev, jnp.zeros((Np, 128), add_prev.dtype)], axis=1)
            both = _mm(A, rhs, addend=ad, out_dtype=bf)
            h2in = both[:, :h_bf.shape[1]]
            indeg = both[:, h_bf.shape[1]:]
        else:
            h2in = _mm(A, h_bf, addend=add_prev, out_dtype=bf)
        t = _mm(h2in, w1.astype(bf), bias=_row8(b1), relu=True, out_dtype=bf)
        z, stats = _mm(t, w2.astype(bf), bias=_row8(b2), stats_n=N)
        h_bf = _bn_relu(z, stats, _row8(lyr["bn_g"]), _row8(lyr["bn_b"]), N)
        add_prev = h_bf

    h128 = _mm(h_bf, params["proj"]["W"].astype(bf),
               bias=_row8(params["proj"]["b"]), out_dtype=bf)
    g = params["gmt"]
    hx_bf = _mm(h128, g["lin1"]["W"].astype(bf),
                bias=_row8(g["lin1"]["b"]), out_dtype=bf)

    p1 = g["pma1"]
    wkv = jnp.concatenate([p1["gcn_k"]["W"], p1["gcn_v"]["W"]], axis=1)
    bkv = jnp.concatenate([p1["gcn_k"]["b"], p1["gcn_v"]["b"]])
    hkv = _mm(hx_bf, wkv.astype(bf))
    u = _scale_dinv(hkv, indeg, N, out_dtype=bf)
    w_agg = _mm(A, u, addend=u)
    kv = _scale_dinv(w_agg, indeg, N, bias8=_row8(bkv))

    starts = jnp.searchsorted(
        batch, jnp.arange(_B + 1, dtype=batch.dtype), side="left").astype(jnp.int32)
    starts = jnp.pad(starts, (0, 72 - (_B + 1)))

    S1 = p1["S"][0]
    n_seeds = S1.shape[0]
    Sp = _rup(n_seeds, 8)
    s1p = jnp.pad(S1, ((0, Sp - n_seeds), (0, 0)))
    bx1 = _pma1(kv, s1p, p1["fc_q"]["W"], _row8(p1["fc_q"]["b"]),
                p1["fc_o"]["W"], _row8(p1["fc_o"]["b"]), starts, n_seeds)

    s = g["sab"]
    p2 = g["pma2"]
    s2p = jnp.pad(p2["S"][0], ((0, 8 - p2["S"].shape[1]), (0, 0)))
    outw = jnp.pad(params["out"]["W"], ((0, 0), (0, D - params["out"]["W"].shape[1])))
    outb = jnp.pad(params["out"]["b"], (0, D - params["out"]["b"].shape[0]))
    y = _gmt_tail(
        bx1, n_seeds,
        (s["fc_q"]["W"], _row8(s["fc_q"]["b"]),
         s["layer_k"]["W"], _row8(s["layer_k"]["b"]),
         s["layer_v"]["W"], _row8(s["layer_v"]["b"]),
         s["fc_o"]["W"], _row8(s["fc_o"]["b"])),
        (p2["fc_q"]["W"], _row8(p2["fc_q"]["b"]),
         p2["layer_k"]["W"], _row8(p2["layer_k"]["b"]),
         p2["layer_v"]["W"], _row8(p2["layer_v"]["b"]),
         p2["fc_o"]["W"], _row8(p2["fc_o"]["b"])),
        s2p, g["lin2"]["W"], _row8(g["lin2"]["b"]), outw, _row8(outb))
    return y[:, 0, :params["out"]["W"].shape[1]]


# speedup vs baseline: 6.1071x; 1.0068x over previous
import math

import jax
import jax.numpy as jnp
from jax.experimental import pallas as pl
from jax.experimental.pallas import tpu as pltpu

_B = 64
_HEADS = 4


def _cdiv(a, b):
    return (a + b - 1) // b


def _rup(a, b):
    return _cdiv(a, b) * b


def _mm(x, y, *, bias=None, addend=None, relu=False, out_dtype=jnp.float32,
        stats_n=None):
    M, K = x.shape
    _, Nn = y.shape
    bm = 512 if M % 512 == 0 else min(256, M)
    bn = min(1024, Nn)
    bk = min(512, K)
    gm, gn, gk = M // bm, Nn // bn, K // bk

    def kern(*refs):
        i = 0
        x_ref, y_ref = refs[0], refs[1]
        nxt = 2
        b_ref = ad_ref = None
        if bias is not None:
            b_ref = refs[nxt]
            nxt += 1
        if addend is not None:
            ad_ref = refs[nxt]
            nxt += 1
        if stats_n is not None:
            o_ref, s_ref, acc_ref = refs[nxt], refs[nxt + 1], refs[nxt + 2]
        else:
            o_ref, acc_ref = refs[nxt], refs[nxt + 1]
        i = pl.program_id(0)
        k = pl.program_id(2)

        @pl.when(k == 0)
        def _():
            acc_ref[...] = jnp.zeros_like(acc_ref)

        acc_ref[...] += jnp.dot(x_ref[...], y_ref[...],
                                preferred_element_type=jnp.float32)

        @pl.when(k == gk - 1)
        def _():
            r = acc_ref[...]
            if b_ref is not None:
                r = r + b_ref[0:1, :]
            if ad_ref is not None:
                r = r + ad_ref[...].astype(jnp.float32)
            if relu:
                r = jnp.maximum(r, 0.0)
            o_ref[...] = r.astype(out_dtype)
            if stats_n is not None:
                @pl.when(i == 0)
                def _():
                    s_ref[...] = jnp.zeros_like(s_ref)

                rows = jax.lax.broadcasted_iota(jnp.int32, (bm, 1), 0) + i * bm
                rm = jnp.where(rows < stats_n, r, 0.0)
                s_ref[0:1, :] += jnp.sum(rm, axis=0, keepdims=True)
                s_ref[1:2, :] += jnp.sum(rm * rm, axis=0, keepdims=True)

    in_specs = [
        pl.BlockSpec((bm, bk), lambda i, j, k: (i, k)),
        pl.BlockSpec((bk, bn), lambda i, j, k: (k, j)),
    ]
    ops = [x, y]
    if bias is not None:
        in_specs.append(pl.BlockSpec((8, bn), lambda i, j, k: (0, j)))
        ops.append(bias)
    if addend is not None:
        in_specs.append(pl.BlockSpec((bm, bn), lambda i, j, k: (i, j)))
        ops.append(addend)
    out_specs = pl.BlockSpec((bm, bn), lambda i, j, k: (i, j))
    out_shape = jax.ShapeDtypeStruct((M, Nn), out_dtype)
    sem = ("parallel", "parallel", "arbitrary")
    if stats_n is not None:
        out_specs = [out_specs, pl.BlockSpec((8, bn), lambda i, j, k: (0, j))]
        out_shape = [out_shape, jax.ShapeDtypeStruct((8, Nn), jnp.float32)]
        sem = ("arbitrary", "parallel", "arbitrary")
    return pl.pallas_call(
        kern,
        grid=(gm, gn, gk),
        in_specs=in_specs,
        out_specs=out_specs,
        out_shape=out_shape,
        scratch_shapes=[pltpu.VMEM((bm, bn), jnp.float32)],
        compiler_params=pltpu.CompilerParams(dimension_semantics=sem),
    )(*ops)


def _colstats(z, n_valid):
    M, Dh = z.shape
    bm = min(512, M)
    gm = M // bm

    def kern(z_ref, o_ref):
        i = pl.program_id(0)

        @pl.when(i == 0)
        def _():
            o_ref[...] = jnp.zeros_like(o_ref)

        rows = jax.lax.broadcasted_iota(jnp.int32, (bm, 1), 0) + i * bm
        zz = jnp.where(rows < n_valid, z_ref[...], 0.0)
        o_ref[0:1, :] += jnp.sum(zz, axis=0, keepdims=True)
        o_ref[1:2, :] += jnp.sum(zz * zz, axis=0, keepdims=True)

    return pl.pallas_call(
        kern,
        grid=(gm,),
        in_specs=[pl.BlockSpec((bm, Dh), lambda i: (i, 0))],
        out_specs=pl.BlockSpec((8, Dh), lambda i: (0, 0)),
        out_shape=jax.ShapeDtypeStruct((8, Dh), jnp.float32),
        compiler_params=pltpu.CompilerParams(
            dimension_semantics=("arbitrary",)),
    )(z)


def _bn_relu(z, stats, g8, b8, n_valid):
    M, Dh = z.shape
    bm = min(512, M)

    def kern(z_ref, s_ref, g_ref, b_ref, o_ref):
        inv_n = 1.0 / n_valid
        mu = s_ref[0:1, :] * inv_n
        var = s_ref[1:2, :] * inv_n - mu * mu
        scale = jax.lax.rsqrt(var + 1e-5) * g_ref[0:1, :]
        r = (z_ref[...] - mu) * scale + b_ref[0:1, :]
        o_ref[...] = jnp.maximum(r, 0.0).astype(jnp.bfloat16)

    return pl.pallas_call(
        kern,
        grid=(M // bm,),
        in_specs=[
            pl.BlockSpec((bm, Dh), lambda i: (i, 0)),
            pl.BlockSpec((8, Dh), lambda i: (0, 0)),
            pl.BlockSpec((8, Dh), lambda i: (0, 0)),
            pl.BlockSpec((8, Dh), lambda i: (0, 0)),
        ],
        out_specs=pl.BlockSpec((bm, Dh), lambda i: (i, 0)),
        out_shape=jax.ShapeDtypeStruct((M, Dh), jnp.bfloat16),
        compiler_params=pltpu.CompilerParams(
            dimension_semantics=("arbitrary",)),
    )(z, stats, g8, b8)


def _scale_dinv(v, indeg, n_valid, bias8=None, out_dtype=jnp.float32):
    M, C = v.shape
    bm = min(512, M)

    def kern(*refs):
        if bias8 is not None:
            v_ref, ind_ref, b_ref, o_ref = refs
        else:
            v_ref, ind_ref, o_ref = refs
            b_ref = None
        i = pl.program_id(0)
        rows = jax.lax.broadcasted_iota(jnp.int32, (bm, 1), 0) + i * bm
        ind = ind_ref[...][:, 0:1].astype(jnp.float32)
        dinv = jnp.where(rows < n_valid, jax.lax.rsqrt(ind + 1.0), 0.0)
        r = v_ref[...].astype(jnp.float32) * dinv
        if b_ref is not None:
            r = r + b_ref[0:1, :]
        o_ref[...] = r.astype(out_dtype)

    in_specs = [
        pl.BlockSpec((bm, C), lambda i: (i, 0)),
        pl.BlockSpec((bm, 128), lambda i: (i, 0)),
    ]
    ops = [v, indeg]
    if bias8 is not None:
        in_specs.append(pl.BlockSpec((8, C), lambda i: (0, 0)))
        ops.append(bias8)
    return pl.pallas_call(
        kern,
        grid=(M // bm,),
        in_specs=in_specs,
        out_specs=pl.BlockSpec((bm, C), lambda i: (i, 0)),
        out_shape=jax.ShapeDtypeStruct((M, C), out_dtype),
        compiler_params=pltpu.CompilerParams(
            dimension_semantics=("arbitrary",)),
    )(*ops)


def _pma1(kv, s1p, wq, bq8, wo, bo8, starts, n_seeds):
    Np, C = kv.shape
    D = C // 2
    hd = D // _HEADS
    Sp, _ = s1p.shape
    T = 1024 if Np % 1024 == 0 else 512
    NT = Np // T
    scale = 1.0 / math.sqrt(D)

    def kern(st_ref, kv_ref, s1_ref, wq_ref, bq_ref, wo_ref, bo_ref,
             o_ref, q_ref, m_ref, l_ref, acc_ref):
        b = pl.program_id(0)
        t = pl.program_id(1)
        start = st_ref[b]
        end = st_ref[b + 1]

        @pl.when(t == 0)
        def _():
            q_ref[...] = jnp.dot(s1_ref[...], wq_ref[...],
                                 preferred_element_type=jnp.float32) + bq_ref[0:1, :]
            m_ref[...] = jnp.full((Sp, D), -1e30, jnp.float32)
            l_ref[...] = jnp.zeros((Sp, D), jnp.float32)
            acc_ref[...] = jnp.zeros((Sp, D), jnp.float32)

        @pl.when((t * T < end) & ((t + 1) * T > start))
        def _():
            kt = kv_ref[pl.ds(t * T, T), :]
            ids = jax.lax.broadcasted_iota(jnp.int32, (1, T), 1) + t * T
            valid = (ids >= start) & (ids < end)
            qp = q_ref[...]
            for h in range(_HEADS):
                lo, hi = h * hd, (h + 1) * hd
                qh = qp[:, lo:hi].astype(jnp.bfloat16)
                kh = kt[:, lo:hi].astype(jnp.bfloat16)
                vh = kt[:, D + lo:D + hi].astype(jnp.bfloat16)
                s = jax.lax.dot_general(qh, kh, (((1,), (1,)), ((), ())),
                                        preferred_element_type=jnp.float32) * scale
                s = jnp.where(valid, s, -1e30)
                mold = m_ref[:, lo:hi][:, 0:1]
                mnew = jnp.maximum(mold, jnp.max(s, axis=1, keepdims=True))
                p = jnp.where(valid, jnp.exp(s - mnew), 0.0)
                resc = jnp.exp(mold - mnew)
                lnew = l_ref[:, lo:hi][:, 0:1] * resc + jnp.sum(p, axis=1, keepdims=True)
                accn = acc_ref[:, lo:hi] * resc + jnp.dot(
                    p.astype(jnp.bfloat16), vh, preferred_element_type=jnp.float32)
                m_ref[:, lo:hi] = jnp.broadcast_to(mnew, (Sp, hd))
                l_ref[:, lo:hi] = jnp.broadcast_to(lnew, (Sp, hd))
                acc_ref[:, lo:hi] = accn

        @pl.when(t == NT - 1)
        def _():
            l = l_ref[...]
            attn = acc_ref[...] * jnp.where(l > 0, 1.0 / l, 0.0)
            o = q_ref[...] + attn
            o2 = jnp.dot(o, wo_ref[...], preferred_element_type=jnp.float32)
            o_ref[0] = o + jnp.maximum(o2 + bo_ref[0:1, :], 0.0)

    return pl.pallas_call(
        kern,
        grid=(_B, NT),
        in_specs=[
            pl.BlockSpec(memory_space=pltpu.SMEM),
            pl.BlockSpec((Np, C), lambda b, t: (0, 0)),
            pl.BlockSpec((Sp, D), lambda b, t: (0, 0)),
            pl.BlockSpec((D, D), lambda b, t: (0, 0)),
            pl.BlockSpec((8, D), lambda b, t: (0, 0)),
            pl.BlockSpec((D, D), lambda b, t: (0, 0)),
            pl.BlockSpec((8, D), lambda b, t: (0, 0)),
        ],
        out_specs=pl.BlockSpec((1, Sp, D), lambda b, t: (b, 0, 0)),
        out_shape=jax.ShapeDtypeStruct((_B, Sp, D), jnp.float32),
        scratch_shapes=[pltpu.VMEM((Sp, D), jnp.float32)] * 4,
        compiler_params=pltpu.CompilerParams(
            dimension_semantics=("arbitrary", "arbitrary")),
    )(starts, kv, s1p, wq, bq8, wo, bo8)


def _attn_block(x, kk, vv, qh_all, wo, bo8, n_seeds, D):
    hd = D // _HEADS
    scale = 1.0 / math.sqrt(D)
    Sp = kk.shape[0]
    cols = jax.lax.broadcasted_iota(jnp.int32, (1, Sp), 1)
    keymask = cols < n_seeds
    outs = []
    for h in range(_HEADS):
        lo, hi = h * hd, (h + 1) * hd
        qh = qh_all[:, lo:hi]
        kh = kk[:, lo:hi]
        vh = vv[:, lo:hi]
        s = jax.lax.dot_general(qh.astype(jnp.bfloat16), kh.astype(jnp.bfloat16),
                                (((1,), (1,)), ((), ())),
                                preferred_element_type=jnp.float32) * scale
        s = jnp.where(keymask, s, -1e30)
        m = jnp.max(s, axis=1, keepdims=True)
        p = jnp.exp(s - m)
        l = jnp.sum(p, axis=1, keepdims=True)
        a = p * (1.0 / l)
        outs.append(qh + jnp.dot(a.astype(jnp.bfloat16), vh.astype(jnp.bfloat16),
                                 preferred_element_type=jnp.float32))
    o = jnp.concatenate(outs, axis=1)
    o2 = jnp.dot(o, wo, preferred_element_type=jnp.float32)
    return o + jnp.maximum(o2 + bo8[0:1, :], 0.0)


def _gmt_tail(bx1, n_seeds, sab, pma2, s2p, lin2w, lin2b8, outw, outb8):
    Bt, Sp, D = bx1.shape

    (swq, sbq8, swk, sbk8, swv, sbv8, swo, sbo8) = sab
    (pwq, pbq8, pwk, pbk8, pwv, pbv8, pwo, pbo8) = pma2

    def kern(x_ref, swq_r, sbq_r, swk_r, sbk_r, swv_r, sbv_r, swo_r, sbo_r,
             s2_r, pwq_r, pbq_r, pwk_r, pbk_r, pwv_r, pbv_r, pwo_r, pbo_r,
             l2w_r, l2b_r, ow_r, ob_r, o_ref):
        x = x_ref[0]
        qp = jnp.dot(x, swq_r[...], preferred_element_type=jnp.float32) + sbq_r[0:1, :]
        kk = jnp.dot(x, swk_r[...], preferred_element_type=jnp.float32) + sbk_r[0:1, :]
        vv = jnp.dot(x, swv_r[...], preferred_element_type=jnp.float32) + sbv_r[0:1, :]
        x2 = _attn_block(x, kk, vv, qp, swo_r[...], sbo_r[...], n_seeds, D)
        q2 = jnp.dot(s2_r[...], pwq_r[...], preferred_element_type=jnp.float32) + pbq_r[0:1, :]
        k2 = jnp.dot(x2, pwk_r[...], preferred_element_type=jnp.float32) + pbk_r[0:1, :]
        v2 = jnp.dot(x2, pwv_r[...], preferred_element_type=jnp.float32) + pbv_r[0:1, :]
        x3 = _attn_block(None, k2, v2, q2, pwo_r[...], pbo_r[...], n_seeds, D)
        gx = jnp.dot(x3, l2w_r[...], preferred_element_type=jnp.float32) + l2b_r[0:1, :]
        y = jnp.dot(gx, ow_r[...], preferred_element_type=jnp.float32) + ob_r[0:1, :]
        o_ref[0] = y

    wb = lambda shape: pl.BlockSpec(shape, lambda b: (0, 0))
    return pl.pallas_call(
        kern,
        grid=(Bt,),
        in_specs=[pl.BlockSpec((1, Sp, D), lambda b: (b, 0, 0))]
        + [wb((D, D)), wb((8, D))] * 4
        + [wb((8, D))]
        + [wb((D, D)), wb((8, D))] * 4
        + [wb((D, D)), wb((8, D)), wb((D, D)), wb((8, D))],
        out_specs=pl.BlockSpec((1, 8, D), lambda b: (b, 0, 0)),
        out_shape=jax.ShapeDtypeStruct((Bt, 8, D), jnp.float32),
        compiler_params=pltpu.CompilerParams(
            dimension_semantics=("arbitrary",)),
    )(bx1, swq, sbq8, swk, sbk8, swv, sbv8, swo, sbo8,
      s2p, pwq, pbq8, pwk, pbk8, pwv, pbv8, pwo, pbo8,
      lin2w, lin2b8, outw, outb8)


def _row8(b):
    return jnp.broadcast_to(b[None, :], (8, b.shape[0])).astype(jnp.float32)


def kernel(x, edge_index, edge_weight, batch, params):
    del edge_weight
    N, D = x.shape
    Np = _rup(N, 2048)
    bf = jnp.bfloat16
    src, dst = edge_index[0], edge_index[1]

    flat = dst.astype(jnp.int32) * Np + src.astype(jnp.int32)
    A = jnp.zeros((Np * Np,), jnp.int32).at[flat].add(1)
    A = A.reshape(Np, Np).astype(bf)

    h_bf = jnp.pad(x, ((0, Np - N), (0, 0))).astype(bf)
    add_prev = jnp.pad(x, ((0, Np - N), (0, 0)))
    indeg = None
    for lyr in params["gin"]:
        w1, b1 = lyr["lin1"]["W"], lyr["lin1"]["b"]
        w2, b2 = lyr["lin2"]["W"], lyr["lin2"]["b"]
        if indeg is None:
            rhs = jnp.concatenate([h_bf, jnp.ones((Np, 128), bf)], axis=1)
            ad = jnp.concatenate(
                [add_prev, jnp.zeros((Np, 128), add_prev.dtype)], axis=1)
            both = _mm(A, rhs, addend=ad, out_dtype=bf)
            h2in = both[:, :h_bf.shape[1]]
            indeg = both[:, h_bf.shape[1]:]
        else:
            h2in = _mm(A, h_bf, addend=add_prev, out_dtype=bf)
        t = _mm(h2in, w1.astype(bf), bias=_row8(b1), relu=True, out_dtype=bf)
        z, stats = _mm(t, w2.astype(bf), bias=_row8(b2), stats_n=N, out_dtype=bf)
        h_bf = _bn_relu(z, stats, _row8(lyr["bn_g"]), _row8(lyr["bn_b"]), N)
        add_prev = h_bf

    h128 = _mm(h_bf, params["proj"]["W"].astype(bf),
               bias=_row8(params["proj"]["b"]), out_dtype=bf)
    g = params["gmt"]
    hx_bf = _mm(h128, g["lin1"]["W"].astype(bf),
                bias=_row8(g["lin1"]["b"]), out_dtype=bf)

    p1 = g["pma1"]
    wkv = jnp.concatenate([p1["gcn_k"]["W"], p1["gcn_v"]["W"]], axis=1)
    bkv = jnp.concatenate([p1["gcn_k"]["b"], p1["gcn_v"]["b"]])
    hkv = _mm(hx_bf, wkv.astype(bf))
    u = _scale_dinv(hkv, indeg, N, out_dtype=bf)
    w_agg = _mm(A, u, addend=u)
    kv = _scale_dinv(w_agg, indeg, N, bias8=_row8(bkv))

    starts = jnp.searchsorted(
        batch, jnp.arange(_B + 1, dtype=batch.dtype), side="left").astype(jnp.int32)
    starts = jnp.pad(starts, (0, 72 - (_B + 1)))

    S1 = p1["S"][0]
    n_seeds = S1.shape[0]
    Sp = _rup(n_seeds, 8)
    s1p = jnp.pad(S1, ((0, Sp - n_seeds), (0, 0)))
    bx1 = _pma1(kv, s1p, p1["fc_q"]["W"], _row8(p1["fc_q"]["b"]),
                p1["fc_o"]["W"], _row8(p1["fc_o"]["b"]), starts, n_seeds)

    s = g["sab"]
    p2 = g["pma2"]
    s2p = jnp.pad(p2["S"][0], ((0, 8 - p2["S"].shape[1]), (0, 0)))
    outw = jnp.pad(params["out"]["W"], ((0, 0), (0, D - params["out"]["W"].shape[1])))
    outb = jnp.pad(params["out"]["b"], (0, D - params["out"]["b"].shape[0]))
    y = _gmt_tail(
        bx1, n_seeds,
        (s["fc_q"]["W"], _row8(s["fc_q"]["b"]),
         s["layer_k"]["W"], _row8(s["layer_k"]["b"]),
         s["layer_v"]["W"], _row8(s["layer_v"]["b"]),
         s["fc_o"]["W"], _row8(s["fc_o"]["b"])),
        (p2["fc_q"]["W"], _row8(p2["fc_q"]["b"]),
         p2["layer_k"]["W"], _row8(p2["layer_k"]["b"]),
         p2["layer_v"]["W"], _row8(p2["layer_v"]["b"]),
         p2["fc_o"]["W"], _row8(p2["fc_o"]["b"])),
        s2p, g["lin2"]["W"], _row8(g["lin2"]["b"]), outw, _row8(outb))
    return y[:, 0, :params["out"]["W"].shape[1]]
